# Initial kernel scaffold; baseline (speedup 1.0000x reference)
import jax
import jax.numpy as jnp
from jax.experimental import pallas as pl


def kernel(hidden_states, gate_w, w_gate, w_up, w_down):
    raise NotImplementedError("write your pallas kernel here")


# speedup vs baseline: 1.5963x; 1.5963x over previous
import functools

import jax
import jax.numpy as jnp
from jax import lax
from jax.experimental import pallas as pl
from jax.experimental.pallas import tpu as pltpu
from jax.experimental.pallas import tpu_sc as plsc

T = 2048
D = 768
F = 512
E = 16
TOPK = 2
BLK = 128
NB = 48
R = NB * BLK
P = T * TOPK

_NC, _NS, _L = 2, 16, 16
NW = _NC * _NS
PPW = P // NW
SCH = 64
TPW = T // NW


def _route_body(x_ref, gw_ref, pos_ref, w_ref, be_ref):
    x = x_ref[...]
    logits = jnp.dot(x, gw_ref[...], preferred_element_type=jnp.float32)
    lane = lax.broadcasted_iota(jnp.int32, (T, E), 1)
    m1 = jnp.max(logits, axis=1, keepdims=True)
    e1 = jnp.min(jnp.where(logits == m1, lane, E), axis=1, keepdims=True)
    masked = jnp.where(lane == e1, -jnp.inf, logits)
    m2 = jnp.max(masked, axis=1, keepdims=True)
    e2 = jnp.min(jnp.where(masked == m2, lane, E), axis=1, keepdims=True)
    w1 = jax.nn.sigmoid(m1 - m2)
    w_ref[...] = jnp.concatenate([w1, 1.0 - w1], axis=1)

    oh1 = (lane == e1).astype(jnp.float32)
    oh2 = (lane == e2).astype(jnp.float32)
    ri = lax.broadcasted_iota(jnp.int32, (T, T), 0)
    ci = lax.broadcasted_iota(jnp.int32, (T, T), 1)
    ltri = (ci <= ri).astype(jnp.float32)
    c1 = jnp.dot(ltri, oh1, preferred_element_type=jnp.float32)
    tot1 = jnp.sum(oh1, axis=0, keepdims=True)
    c2 = jnp.dot(ltri, oh2, preferred_element_type=jnp.float32) + tot1
    tot = tot1 + jnp.sum(oh2, axis=0, keepdims=True)
    nblk = jnp.floor((tot + (BLK - 1)) / BLK)
    si = lax.broadcasted_iota(jnp.int32, (E, E), 0)
    sj = lax.broadcasted_iota(jnp.int32, (E, E), 1)
    stri = (si < sj).astype(jnp.float32)
    bstart = jnp.dot(nblk, stri, preferred_element_type=jnp.float32)
    po = bstart * BLK
    rank0 = jnp.sum(oh1 * c1, axis=1, keepdims=True) - 1.0
    rank1 = jnp.sum(oh2 * c2, axis=1, keepdims=True) - 1.0
    po1 = jnp.sum(oh1 * po, axis=1, keepdims=True)
    po2 = jnp.sum(oh2 * po, axis=1, keepdims=True)
    pos0 = (po1 + rank0).astype(jnp.int32)
    pos1 = (po2 + rank1).astype(jnp.int32)
    pos_ref[...] = jnp.concatenate([pos0, pos1], axis=1)
    bi = lax.broadcasted_iota(jnp.int32, (NB, E), 0).astype(jnp.float32)
    ind = (bi >= bstart).astype(jnp.float32)
    be_ref[...] = (jnp.sum(ind, axis=1, keepdims=True) - 1.0).astype(jnp.int32)


def _route(x, gate_w):
    return pl.pallas_call(
        _route_body,
        out_shape=(
            jax.ShapeDtypeStruct((T, TOPK), jnp.int32),
            jax.ShapeDtypeStruct((T, TOPK), jnp.float32),
            jax.ShapeDtypeStruct((NB, 1), jnp.int32),
        ),
    )(x, gate_w)


def _ffn_body(be_ref, x_ref, wg_ref, wu_ref, wd_ref, y_ref):
    xb = x_ref[...]
    g = jnp.dot(xb, wg_ref[0], preferred_element_type=jnp.float32)
    u = jnp.dot(xb, wu_ref[0], preferred_element_type=jnp.float32)
    h = g * jax.nn.sigmoid(g) * u
    y_ref[...] = jnp.dot(h, wd_ref[0], preferred_element_type=jnp.float32)


def _ffn(be, xs, w_gate, w_up, w_down):
    grid_spec = pltpu.PrefetchScalarGridSpec(
        num_scalar_prefetch=1,
        grid=(NB,),
        in_specs=[
            pl.BlockSpec((BLK, D), lambda i, be: (i, 0)),
            pl.BlockSpec((1, D, F), lambda i, be: (be[i], 0, 0)),
            pl.BlockSpec((1, D, F), lambda i, be: (be[i], 0, 0)),
            pl.BlockSpec((1, F, D), lambda i, be: (be[i], 0, 0)),
        ],
        out_specs=pl.BlockSpec((BLK, D), lambda i, be: (i, 0)),
    )
    return pl.pallas_call(
        _ffn_body,
        grid_spec=grid_spec,
        out_shape=jax.ShapeDtypeStruct((R, D), jnp.float32),
    )(be, xs, w_gate, w_up, w_down)


@functools.cache
def _get_scatter_x():
    mesh = plsc.VectorSubcoreMesh(core_axis_name="c", subcore_axis_name="s")
    return pl.kernel(
        _scatter_x_body,
        mesh=mesh,
        out_type=jax.ShapeDtypeStruct((R, D), jnp.float32),
        scratch_types=[
            pltpu.VMEM((SCH,), jnp.int32),
            pltpu.VMEM((SCH, D), jnp.float32),
            pltpu.SemaphoreType.DMA,
        ],
    )


def _scatter_x_body(x_hbm, posf_hbm, xs_hbm, idx_v, rows_v, sem):
    wid = lax.axis_index("c") * _NS + lax.axis_index("s")
    pbase = wid * PPW
    for ch in range(PPW // SCH):
        off = pbase + ch * SCH
        tok = off % T
        pltpu.sync_copy(posf_hbm.at[pl.ds(off, SCH)], idx_v)
        pltpu.sync_copy(x_hbm.at[pl.ds(tok, SCH)], rows_v)
        pltpu.async_copy(rows_v, xs_hbm.at[idx_v], sem).wait()


@functools.cache
def _get_combine():
    mesh = plsc.VectorSubcoreMesh(core_axis_name="c", subcore_axis_name="s")
    return pl.kernel(
        _combine_body,
        mesh=mesh,
        out_type=jax.ShapeDtypeStruct((T, D), jnp.float32),
        scratch_types=[
            pltpu.VMEM((TPW,), jnp.int32),
            pltpu.VMEM((TPW,), jnp.int32),
            pltpu.VMEM((TPW,), jnp.float32),
            pltpu.VMEM((TPW,), jnp.float32),
            pltpu.VMEM((TPW, D), jnp.float32),
            pltpu.VMEM((TPW, D), jnp.float32),
            pltpu.SemaphoreType.DMA,
        ],
    )


def _combine_body(y_hbm, posf_hbm, wf_hbm, out_hbm, i0_v, i1_v, w0_v, w1_v,
                  r0_v, r1_v, sem):
    wid = lax.axis_index("c") * _NS + lax.axis_index("s")
    tb = wid * TPW
    pltpu.sync_copy(posf_hbm.at[pl.ds(tb, TPW)], i0_v)
    pltpu.sync_copy(posf_hbm.at[pl.ds(T + tb, TPW)], i1_v)
    pltpu.sync_copy(wf_hbm.at[pl.ds(tb, TPW)], w0_v)
    pltpu.sync_copy(wf_hbm.at[pl.ds(T + tb, TPW)], w1_v)
    pltpu.async_copy(y_hbm.at[i0_v], r0_v, sem).wait()
    pltpu.async_copy(y_hbm.at[i1_v], r1_v, sem).wait()

    for g in range(TPW // _L):
        wv0 = w0_v[pl.ds(g * _L, _L)]
        wv1 = w1_v[pl.ds(g * _L, _L)]

        def body(lane, carry, wv0=wv0, wv1=wv1, g=g):
            iv = jnp.full((_L,), lane, jnp.int32)
            w0s = wv0.at[iv].get(mode="promise_in_bounds")
            w1s = wv1.at[iv].get(mode="promise_in_bounds")
            i = g * _L + lane
            for j in range(D // _L):
                sl = pl.ds(j * _L, _L)
                r0_v[i, sl] = w0s * r0_v[i, sl] + w1s * r1_v[i, sl]
            return carry

        lax.fori_loop(0, _L, body, 0)
    pltpu.sync_copy(r0_v, out_hbm.at[pl.ds(tb, TPW)])


def kernel(hidden_states, gate_w, w_gate, w_up, w_down):
    b, s, d = hidden_states.shape
    x = hidden_states.reshape(-1, d)
    pos, w, be = _route(x, gate_w)
    posf = jnp.concatenate([pos[:, 0], pos[:, 1]])
    wf = jnp.concatenate([w[:, 0], w[:, 1]])
    xs = _get_scatter_x()(x, posf)
    ys = _ffn(be.reshape(NB), xs, w_gate, w_up, w_down)
    out = _get_combine()(ys, posf, wf)
    return out.reshape(b, s, d)


# speedup vs baseline: 1.6170x; 1.0130x over previous
import functools

import jax
import jax.numpy as jnp
from jax import lax
from jax.experimental import pallas as pl
from jax.experimental.pallas import tpu as pltpu
from jax.experimental.pallas import tpu_sc as plsc

T = 2048
D = 768
F = 512
E = 16
TOPK = 2
BLK = 128
NB = 48
R = NB * BLK
P = T * TOPK

_NC, _NS, _L = 2, 16, 16
NW = _NC * _NS
SCH = T // _NS
TPW = T // NW


def _route_body(x_ref, gw_ref, p0_ref, p1_ref, w0_ref, w1_ref, be_ref):
    x = x_ref[...]
    logits = jnp.dot(x, gw_ref[...], preferred_element_type=jnp.float32)
    lane = lax.broadcasted_iota(jnp.int32, (T, E), 1)
    m1 = jnp.max(logits, axis=1, keepdims=True)
    e1 = jnp.min(jnp.where(logits == m1, lane, E), axis=1, keepdims=True)
    masked = jnp.where(lane == e1, -jnp.inf, logits)
    m2 = jnp.max(masked, axis=1, keepdims=True)
    e2 = jnp.min(jnp.where(masked == m2, lane, E), axis=1, keepdims=True)
    w1 = jax.nn.sigmoid(m1 - m2)
    w0_ref[...] = w1
    w1_ref[...] = 1.0 - w1

    oh1 = (lane == e1).astype(jnp.float32)
    oh2 = (lane == e2).astype(jnp.float32)
    ri = lax.broadcasted_iota(jnp.int32, (T, T), 0)
    ci = lax.broadcasted_iota(jnp.int32, (T, T), 1)
    ltri = (ci <= ri).astype(jnp.float32)
    c1 = jnp.dot(ltri, oh1, preferred_element_type=jnp.float32)
    tot1 = jnp.sum(oh1, axis=0, keepdims=True)
    c2 = jnp.dot(ltri, oh2, preferred_element_type=jnp.float32) + tot1
    tot = tot1 + jnp.sum(oh2, axis=0, keepdims=True)
    nblk = jnp.floor((tot + (BLK - 1)) / BLK)
    si = lax.broadcasted_iota(jnp.int32, (E, E), 0)
    sj = lax.broadcasted_iota(jnp.int32, (E, E), 1)
    stri = (si < sj).astype(jnp.float32)
    bstart = jnp.dot(nblk, stri, preferred_element_type=jnp.float32)
    po = bstart * BLK
    rank0 = jnp.sum(oh1 * c1, axis=1, keepdims=True) - 1.0
    rank1 = jnp.sum(oh2 * c2, axis=1, keepdims=True) - 1.0
    po1 = jnp.sum(oh1 * po, axis=1, keepdims=True)
    po2 = jnp.sum(oh2 * po, axis=1, keepdims=True)
    p0_ref[...] = (po1 + rank0).astype(jnp.int32)
    p1_ref[...] = (po2 + rank1).astype(jnp.int32)
    bi = lax.broadcasted_iota(jnp.int32, (NB, E), 0).astype(jnp.float32)
    ind = (bi >= bstart).astype(jnp.float32)
    be_ref[...] = (jnp.sum(ind, axis=1, keepdims=True) - 1.0).astype(jnp.int32)


def _route(x, gate_w):
    return pl.pallas_call(
        _route_body,
        out_shape=(
            jax.ShapeDtypeStruct((T, 1), jnp.int32),
            jax.ShapeDtypeStruct((T, 1), jnp.int32),
            jax.ShapeDtypeStruct((T, 1), jnp.float32),
            jax.ShapeDtypeStruct((T, 1), jnp.float32),
            jax.ShapeDtypeStruct((NB, 1), jnp.int32),
        ),
    )(x, gate_w)


def _ffn_body(be_ref, x_ref, wg_ref, wu_ref, wd_ref, y_ref):
    xb = x_ref[...]
    g = jnp.dot(xb, wg_ref[0], preferred_element_type=jnp.float32)
    u = jnp.dot(xb, wu_ref[0], preferred_element_type=jnp.float32)
    h = g * jax.nn.sigmoid(g) * u
    y_ref[...] = jnp.dot(h, wd_ref[0], preferred_element_type=jnp.float32)


def _ffn(be, xs, w_gate, w_up, w_down):
    grid_spec = pltpu.PrefetchScalarGridSpec(
        num_scalar_prefetch=1,
        grid=(NB,),
        in_specs=[
            pl.BlockSpec((BLK, D), lambda i, be: (i, 0)),
            pl.BlockSpec((1, D, F), lambda i, be: (be[i], 0, 0)),
            pl.BlockSpec((1, D, F), lambda i, be: (be[i], 0, 0)),
            pl.BlockSpec((1, F, D), lambda i, be: (be[i], 0, 0)),
        ],
        out_specs=pl.BlockSpec((BLK, D), lambda i, be: (i, 0)),
    )
    return pl.pallas_call(
        _ffn_body,
        grid_spec=grid_spec,
        out_shape=jax.ShapeDtypeStruct((R, D), jnp.float32),
    )(be, xs, w_gate, w_up, w_down)


@functools.cache
def _get_scatter_x():
    mesh = plsc.VectorSubcoreMesh(core_axis_name="c", subcore_axis_name="s")
    return pl.kernel(
        _scatter_x_body,
        mesh=mesh,
        out_type=jax.ShapeDtypeStruct((R, D), jnp.float32),
        scratch_types=[
            pltpu.VMEM((SCH,), jnp.int32),
            pltpu.VMEM((SCH, D), jnp.float32),
            pltpu.SemaphoreType.DMA,
        ],
    )


def _scatter_x_body(x_hbm, p0_hbm, p1_hbm, xs_hbm, idx_v, rows_v, sem):
    k = lax.axis_index("c")
    t0 = lax.axis_index("s") * SCH

    @pl.when(k == 0)
    def _():
        pltpu.sync_copy(p0_hbm.at[pl.ds(t0, SCH)], idx_v)

    @pl.when(k == 1)
    def _():
        pltpu.sync_copy(p1_hbm.at[pl.ds(t0, SCH)], idx_v)

    pltpu.sync_copy(x_hbm.at[pl.ds(t0, SCH)], rows_v)
    pltpu.async_copy(rows_v, xs_hbm.at[idx_v], sem).wait()


@functools.cache
def _get_combine():
    mesh = plsc.VectorSubcoreMesh(core_axis_name="c", subcore_axis_name="s")
    return pl.kernel(
        _combine_body,
        mesh=mesh,
        out_type=jax.ShapeDtypeStruct((T, D), jnp.float32),
        scratch_types=[
            pltpu.VMEM((TPW,), jnp.int32),
            pltpu.VMEM((TPW,), jnp.int32),
            pltpu.VMEM((TPW,), jnp.float32),
            pltpu.VMEM((TPW,), jnp.float32),
            pltpu.VMEM((TPW, D), jnp.float32),
            pltpu.VMEM((TPW, D), jnp.float32),
            pltpu.SemaphoreType.DMA,
        ],
    )


def _combine_body(y_hbm, p0_hbm, p1_hbm, w0_hbm, w1_hbm, out_hbm,
                  i0_v, i1_v, w0_v, w1_v, r0_v, r1_v, sem):
    wid = lax.axis_index("c") * _NS + lax.axis_index("s")
    tb = wid * TPW
    pltpu.sync_copy(p0_hbm.at[pl.ds(tb, TPW)], i0_v)
    pltpu.sync_copy(p1_hbm.at[pl.ds(tb, TPW)], i1_v)
    pltpu.sync_copy(w0_hbm.at[pl.ds(tb, TPW)], w0_v)
    pltpu.sync_copy(w1_hbm.at[pl.ds(tb, TPW)], w1_v)
    c0 = pltpu.async_copy(y_hbm.at[i0_v], r0_v, sem)
    c1 = pltpu.async_copy(y_hbm.at[i1_v], r1_v, sem)
    c0.wait()
    c1.wait()

    for g in range(TPW // _L):
        wv0 = w0_v[pl.ds(g * _L, _L)]
        wv1 = w1_v[pl.ds(g * _L, _L)]

        def body(lane, carry, wv0=wv0, wv1=wv1, g=g):
            iv = jnp.full((_L,), lane, jnp.int32)
            w0s = wv0.at[iv].get(mode="promise_in_bounds")
            w1s = wv1.at[iv].get(mode="promise_in_bounds")
            i = g * _L + lane
            for j in range(D // _L):
                sl = pl.ds(j * _L, _L)
                r0_v[i, sl] = w0s * r0_v[i, sl] + w1s * r1_v[i, sl]
            return carry

        lax.fori_loop(0, _L, body, 0, unroll=2)

    pltpu.sync_copy(r0_v, out_hbm.at[pl.ds(tb, TPW)])


def kernel(hidden_states, gate_w, w_gate, w_up, w_down):
    b, s, d = hidden_states.shape
    x = hidden_states.reshape(-1, d)
    p0, p1, w0, w1, be = _route(x, gate_w)
    p0, p1 = p0.reshape(T), p1.reshape(T)
    w0, w1 = w0.reshape(T), w1.reshape(T)
    xs = _get_scatter_x()(x, p0, p1)
    ys = _ffn(be.reshape(NB), xs, w_gate, w_up, w_down)
    out = _get_combine()(ys, p0, p1, w0, w1)
    return out.reshape(b, s, d)


# speedup vs baseline: 1.6590x; 1.0260x over previous
import functools

import jax
import jax.numpy as jnp
from jax import lax
from jax.experimental import pallas as pl
from jax.experimental.pallas import tpu as pltpu
from jax.experimental.pallas import tpu_sc as plsc

T = 2048
D = 768
F = 512
E = 16
TOPK = 2
BLK = 128
NB = 48
R = NB * BLK
P = T * TOPK

_NC, _NS, _L = 2, 16, 16
NW = _NC * _NS
SCH = T // _NS
TPW = T // NW
CCH = TPW // 2


def _route_body(x_ref, gw_ref, p0_ref, p1_ref, w0_ref, w1_ref, be_ref, bx_ref):
    x = x_ref[...]
    logits = jnp.dot(x, gw_ref[...], preferred_element_type=jnp.float32)
    lane = lax.broadcasted_iota(jnp.int32, (T, E), 1)
    m1 = jnp.max(logits, axis=1, keepdims=True)
    e1 = jnp.min(jnp.where(logits == m1, lane, E), axis=1, keepdims=True)
    masked = jnp.where(lane == e1, -jnp.inf, logits)
    m2 = jnp.max(masked, axis=1, keepdims=True)
    e2 = jnp.min(jnp.where(masked == m2, lane, E), axis=1, keepdims=True)
    w1 = jax.nn.sigmoid(m1 - m2)
    w0_ref[...] = w1
    w1_ref[...] = 1.0 - w1

    oh1 = (lane == e1).astype(jnp.float32)
    oh2 = (lane == e2).astype(jnp.float32)
    ri = lax.broadcasted_iota(jnp.int32, (T, T), 0)
    ci = lax.broadcasted_iota(jnp.int32, (T, T), 1)
    ltri = (ci <= ri).astype(jnp.float32)
    c1 = jnp.dot(ltri, oh1, preferred_element_type=jnp.float32)
    tot1 = jnp.sum(oh1, axis=0, keepdims=True)
    c2 = jnp.dot(ltri, oh2, preferred_element_type=jnp.float32) + tot1
    tot = tot1 + jnp.sum(oh2, axis=0, keepdims=True)
    nblk = jnp.floor((tot + (BLK - 1)) / BLK)
    si = lax.broadcasted_iota(jnp.int32, (E, E), 0)
    sj = lax.broadcasted_iota(jnp.int32, (E, E), 1)
    stri = (si < sj).astype(jnp.float32)
    bstart = jnp.dot(nblk, stri, preferred_element_type=jnp.float32)
    po = bstart * BLK
    rank0 = jnp.sum(oh1 * c1, axis=1, keepdims=True) - 1.0
    rank1 = jnp.sum(oh2 * c2, axis=1, keepdims=True) - 1.0
    po1 = jnp.sum(oh1 * po, axis=1, keepdims=True)
    po2 = jnp.sum(oh2 * po, axis=1, keepdims=True)
    p0_ref[...] = (po1 + rank0).astype(jnp.int32)
    p1_ref[...] = (po2 + rank1).astype(jnp.int32)
    tot = jnp.sum(nblk, keepdims=True)
    bif = lax.broadcasted_iota(jnp.int32, (NB, 1), 0).astype(jnp.float32)
    bxf = jnp.minimum(bif, tot - 1.0)
    bx_ref[...] = bxf.astype(jnp.int32)
    ind = (bxf >= bstart).astype(jnp.float32)
    be_ref[...] = (jnp.sum(ind, axis=1, keepdims=True) - 1.0).astype(jnp.int32)


def _route(x, gate_w):
    return pl.pallas_call(
        _route_body,
        out_shape=(
            jax.ShapeDtypeStruct((T, 1), jnp.int32),
            jax.ShapeDtypeStruct((T, 1), jnp.int32),
            jax.ShapeDtypeStruct((T, 1), jnp.float32),
            jax.ShapeDtypeStruct((T, 1), jnp.float32),
            jax.ShapeDtypeStruct((NB, 1), jnp.int32),
            jax.ShapeDtypeStruct((NB, 1), jnp.int32),
        ),
    )(x, gate_w)


def _ffn_body(be_ref, bx_ref, x_ref, wg_ref, wu_ref, wd_ref, y_ref):
    xb = x_ref[...]
    g = jnp.dot(xb, wg_ref[0], preferred_element_type=jnp.float32)
    u = jnp.dot(xb, wu_ref[0], preferred_element_type=jnp.float32)
    h = g * jax.nn.sigmoid(g) * u
    y_ref[...] = jnp.dot(h, wd_ref[0], preferred_element_type=jnp.float32)


def _ffn(be, bx, xs, w_gate, w_up, w_down):
    grid_spec = pltpu.PrefetchScalarGridSpec(
        num_scalar_prefetch=2,
        grid=(NB,),
        in_specs=[
            pl.BlockSpec((BLK, D), lambda i, be, bx: (bx[i], 0)),
            pl.BlockSpec((1, D, F), lambda i, be, bx: (be[i], 0, 0)),
            pl.BlockSpec((1, D, F), lambda i, be, bx: (be[i], 0, 0)),
            pl.BlockSpec((1, F, D), lambda i, be, bx: (be[i], 0, 0)),
        ],
        out_specs=pl.BlockSpec((BLK, D), lambda i, be, bx: (bx[i], 0)),
    )
    return pl.pallas_call(
        _ffn_body,
        grid_spec=grid_spec,
        out_shape=jax.ShapeDtypeStruct((R, D), jnp.float32),
    )(be, bx, xs, w_gate, w_up, w_down)


@functools.cache
def _get_scatter_x():
    mesh = plsc.VectorSubcoreMesh(core_axis_name="c", subcore_axis_name="s")
    return pl.kernel(
        _scatter_x_body,
        mesh=mesh,
        out_type=jax.ShapeDtypeStruct((R, D), jnp.float32),
        scratch_types=[
            pltpu.VMEM((SCH,), jnp.int32),
            pltpu.VMEM((SCH, D), jnp.float32),
            pltpu.SemaphoreType.DMA,
        ],
    )


def _scatter_x_body(x_hbm, p0_hbm, p1_hbm, xs_hbm, idx_v, rows_v, sem):
    k = lax.axis_index("c")
    t0 = lax.axis_index("s") * SCH

    @pl.when(k == 0)
    def _():
        pltpu.sync_copy(p0_hbm.at[pl.ds(t0, SCH)], idx_v)

    @pl.when(k == 1)
    def _():
        pltpu.sync_copy(p1_hbm.at[pl.ds(t0, SCH)], idx_v)

    pltpu.sync_copy(x_hbm.at[pl.ds(t0, SCH)], rows_v)
    pltpu.async_copy(rows_v, xs_hbm.at[idx_v], sem).wait()


@functools.cache
def _get_combine():
    mesh = plsc.VectorSubcoreMesh(core_axis_name="c", subcore_axis_name="s")
    return pl.kernel(
        _combine_body,
        mesh=mesh,
        out_type=jax.ShapeDtypeStruct((T, D), jnp.float32),
        scratch_types=[
            pltpu.VMEM((CCH,), jnp.int32),
            pltpu.VMEM((CCH,), jnp.int32),
            pltpu.VMEM((CCH,), jnp.int32),
            pltpu.VMEM((CCH,), jnp.int32),
            pltpu.VMEM((TPW,), jnp.float32),
            pltpu.VMEM((TPW,), jnp.float32),
            pltpu.VMEM((CCH, D), jnp.float32),
            pltpu.VMEM((CCH, D), jnp.float32),
            pltpu.VMEM((CCH, D), jnp.float32),
            pltpu.VMEM((CCH, D), jnp.float32),
            pltpu.SemaphoreType.DMA,
            pltpu.SemaphoreType.DMA,
        ],
    )


def _combine_body(y_hbm, p0_hbm, p1_hbm, w0_hbm, w1_hbm, out_hbm,
                  i0a_v, i1a_v, i0b_v, i1b_v, w0_v, w1_v,
                  r0a_v, r1a_v, r0b_v, r1b_v, sem_a, sem_b):
    wid = lax.axis_index("c") * _NS + lax.axis_index("s")
    tb = wid * TPW
    pltpu.sync_copy(w0_hbm.at[pl.ds(tb, TPW)], w0_v)
    pltpu.sync_copy(w1_hbm.at[pl.ds(tb, TPW)], w1_v)
    pltpu.sync_copy(p0_hbm.at[pl.ds(tb, CCH)], i0a_v)
    pltpu.sync_copy(p1_hbm.at[pl.ds(tb, CCH)], i1a_v)
    pltpu.sync_copy(p0_hbm.at[pl.ds(tb + CCH, CCH)], i0b_v)
    pltpu.sync_copy(p1_hbm.at[pl.ds(tb + CCH, CCH)], i1b_v)
    a0 = pltpu.async_copy(y_hbm.at[i0a_v], r0a_v, sem_a)
    a1 = pltpu.async_copy(y_hbm.at[i1a_v], r1a_v, sem_a)
    b0 = pltpu.async_copy(y_hbm.at[i0b_v], r0b_v, sem_b)
    b1 = pltpu.async_copy(y_hbm.at[i1b_v], r1b_v, sem_b)

    def weighted_sum(r0_v, r1_v, gbase):
        for g in range(CCH // _L):
            wv0 = w0_v[pl.ds((gbase + g) * _L, _L)]
            wv1 = w1_v[pl.ds((gbase + g) * _L, _L)]

            def body(lane, carry, wv0=wv0, wv1=wv1, g=g):
                iv = jnp.full((_L,), lane, jnp.int32)
                w0s = wv0.at[iv].get(mode="promise_in_bounds")
                w1s = wv1.at[iv].get(mode="promise_in_bounds")
                i = g * _L + lane
                for j in range(D // _L):
                    sl = pl.ds(j * _L, _L)
                    r0_v[i, sl] = w0s * r0_v[i, sl] + w1s * r1_v[i, sl]
                return carry

            lax.fori_loop(0, _L, body, 0, unroll=2)

    a0.wait()
    a1.wait()
    weighted_sum(r0a_v, r1a_v, 0)
    st_a = pltpu.async_copy(r0a_v, out_hbm.at[pl.ds(tb, CCH)], sem_a)
    b0.wait()
    b1.wait()
    weighted_sum(r0b_v, r1b_v, CCH // _L)
    st_a.wait()
    pltpu.sync_copy(r0b_v, out_hbm.at[pl.ds(tb + CCH, CCH)])


def kernel(hidden_states, gate_w, w_gate, w_up, w_down):
    b, s, d = hidden_states.shape
    x = hidden_states.reshape(-1, d)
    p0, p1, w0, w1, be, bx = _route(x, gate_w)
    p0, p1 = p0.reshape(T), p1.reshape(T)
    w0, w1 = w0.reshape(T), w1.reshape(T)
    xs = _get_scatter_x()(x, p0, p1)
    ys = _ffn(be.reshape(NB), bx.reshape(NB), xs, w_gate, w_up, w_down)
    out = _get_combine()(ys, p0, p1, w0, w1)
    return out.reshape(b, s, d)


# speedup vs baseline: 1.6940x; 1.0211x over previous
import functools

import jax
import jax.numpy as jnp
from jax import lax
from jax.experimental import pallas as pl
from jax.experimental.pallas import tpu as pltpu
from jax.experimental.pallas import tpu_sc as plsc

T = 2048
D = 768
F = 512
E = 16
TOPK = 2
BLK = 128
NB = 48
R = NB * BLK
P = T * TOPK

_NC, _NS, _L = 2, 16, 16
NW = _NC * _NS
SCH = T // _NS
TPW = T // NW
CCH = TPW // 2


def _route_body(x_ref, gw_ref, p0_ref, p1_ref, w0_ref, w1_ref, tbl_ref):
    x = x_ref[...]
    logits = jnp.dot(x, gw_ref[...], preferred_element_type=jnp.float32)
    lane = lax.broadcasted_iota(jnp.int32, (T, E), 1)
    m1 = jnp.max(logits, axis=1, keepdims=True)
    e1 = jnp.min(jnp.where(logits == m1, lane, E), axis=1, keepdims=True)
    masked = jnp.where(lane == e1, -jnp.inf, logits)
    m2 = jnp.max(masked, axis=1, keepdims=True)
    e2 = jnp.min(jnp.where(masked == m2, lane, E), axis=1, keepdims=True)
    w1 = jax.nn.sigmoid(m1 - m2)
    w0_ref[...] = w1
    w1_ref[...] = 1.0 - w1

    oh1 = (lane == e1).astype(jnp.float32)
    oh2 = (lane == e2).astype(jnp.float32)
    ri = lax.broadcasted_iota(jnp.int32, (T, T), 0)
    ci = lax.broadcasted_iota(jnp.int32, (T, T), 1)
    ltri = (ci <= ri).astype(jnp.float32)
    c1 = jnp.dot(ltri, oh1, preferred_element_type=jnp.float32)
    tot1 = jnp.sum(oh1, axis=0, keepdims=True)
    c2 = jnp.dot(ltri, oh2, preferred_element_type=jnp.float32) + tot1
    tot = tot1 + jnp.sum(oh2, axis=0, keepdims=True)
    nblk = jnp.floor((tot + (BLK - 1)) / BLK)
    si = lax.broadcasted_iota(jnp.int32, (E, E), 0)
    sj = lax.broadcasted_iota(jnp.int32, (E, E), 1)
    stri = (si < sj).astype(jnp.float32)
    bstart = jnp.dot(nblk, stri, preferred_element_type=jnp.float32)
    po = bstart * BLK
    rank0 = jnp.sum(oh1 * c1, axis=1, keepdims=True) - 1.0
    rank1 = jnp.sum(oh2 * c2, axis=1, keepdims=True) - 1.0
    po1 = jnp.sum(oh1 * po, axis=1, keepdims=True)
    po2 = jnp.sum(oh2 * po, axis=1, keepdims=True)
    p0_ref[...] = (po1 + rank0).astype(jnp.int32)
    p1_ref[...] = (po2 + rank1).astype(jnp.int32)
    tot = jnp.sum(nblk, keepdims=True)
    bif = lax.broadcasted_iota(jnp.int32, (NB, 1), 0).astype(jnp.float32)
    active = (nblk > 0.0)
    ind = (bif >= bstart).astype(jnp.float32)
    be = jnp.sum(ind, axis=1, keepdims=True) - 1.0
    chg = jnp.sum(((bstart == bif) & active).astype(jnp.float32), axis=1,
                  keepdims=True)
    runcnt = jnp.sum(((bstart <= bif) & active).astype(jnp.float32), axis=1,
                     keepdims=True)
    par = runcnt - 1.0 - 2.0 * jnp.floor((runcnt - 1.0) * 0.5)
    bigv = jnp.float32(1e9)
    startsf = jnp.where(active, bstart, bigv)
    cand = jnp.where(startsf > bif, startsf, bigv)
    nmin = jnp.min(cand, axis=1, keepdims=True)
    hn = (nmin < 1e8).astype(jnp.float32)
    eiota = lax.broadcasted_iota(jnp.int32, (NB, E), 1).astype(jnp.float32)
    nxt = jnp.sum(jnp.where(cand == nmin, eiota, 0.0), axis=1, keepdims=True)
    totb = jnp.zeros((NB, 1), jnp.float32) + tot
    tbl = jnp.concatenate([be, chg, par, nxt, hn, totb,
                           jnp.zeros((NB, 2), jnp.float32)], axis=1)
    tbl_ref[...] = tbl.astype(jnp.int32)


def _route(x, gate_w):
    return pl.pallas_call(
        _route_body,
        out_shape=(
            jax.ShapeDtypeStruct((T, 1), jnp.int32),
            jax.ShapeDtypeStruct((T, 1), jnp.int32),
            jax.ShapeDtypeStruct((T, 1), jnp.float32),
            jax.ShapeDtypeStruct((T, 1), jnp.float32),
            jax.ShapeDtypeStruct((NB, 8), jnp.int32),
        ),
    )(x, gate_w)


def _ffn_body(tbl_ref, xs_ref, wg_ref, wu_ref, wd_ref, y_ref,
              xbuf, wgbuf, wubuf, wdbuf, ybuf, sx, sw, sy):
    tot = tbl_ref[0, 5]

    def wcopies(e, slot):
        return (pltpu.make_async_copy(wg_ref.at[e], wgbuf.at[slot], sw),
                pltpu.make_async_copy(wu_ref.at[e], wubuf.at[slot], sw),
                pltpu.make_async_copy(wd_ref.at[e], wdbuf.at[slot], sw))

    def xcopy(i, slot):
        return pltpu.make_async_copy(
            xs_ref.at[pl.ds(i * BLK, BLK)], xbuf.at[slot], sx)

    def ycopy(i, slot):
        return pltpu.make_async_copy(
            ybuf.at[slot], y_ref.at[pl.ds(i * BLK, BLK)], sy)

    xcopy(0, 0).start()
    for c in wcopies(tbl_ref[0, 0], 0):
        c.start()

    def step(i, carry):
        q = lax.rem(i, 2)
        bev = tbl_ref[i, 0]
        chg = tbl_ref[i, 1]
        par = tbl_ref[i, 2]
        nxt = tbl_ref[i, 3]
        hn = tbl_ref[i, 4]

        @pl.when(chg == 1)
        def _():
            for c in wcopies(bev, par):
                c.wait()

            @pl.when(hn == 1)
            def _():
                for c in wcopies(nxt, 1 - par):
                    c.start()

        xcopy(i, q).wait()

        @pl.when(i + 1 < tot)
        def _():
            xcopy(i + 1, 1 - q).start()

        @pl.when(i >= 2)
        def _():
            ycopy(i, q).wait()

        xb = xbuf[q]
        g = jnp.dot(xb, wgbuf[par], preferred_element_type=jnp.float32)
        u = jnp.dot(xb, wubuf[par], preferred_element_type=jnp.float32)
        h = g * jax.nn.sigmoid(g) * u
        ybuf[q, ...] = jnp.dot(h, wdbuf[par], preferred_element_type=jnp.float32)
        ycopy(i, q).start()
        return carry

    lax.fori_loop(0, tot, step, 0)
    ycopy(0, 0).wait()
    ycopy(0, 1).wait()


def _ffn(tbl, xs, w_gate, w_up, w_down):
    grid_spec = pltpu.PrefetchScalarGridSpec(
        num_scalar_prefetch=1,
        grid=(1,),
        in_specs=[
            pl.BlockSpec(memory_space=pl.ANY),
            pl.BlockSpec(memory_space=pl.ANY),
            pl.BlockSpec(memory_space=pl.ANY),
            pl.BlockSpec(memory_space=pl.ANY),
        ],
        out_specs=pl.BlockSpec(memory_space=pl.ANY),
        scratch_shapes=[
            pltpu.VMEM((2, BLK, D), jnp.float32),
            pltpu.VMEM((2, D, F), jnp.float32),
            pltpu.VMEM((2, D, F), jnp.float32),
            pltpu.VMEM((2, F, D), jnp.float32),
            pltpu.VMEM((2, BLK, D), jnp.float32),
            pltpu.SemaphoreType.DMA,
            pltpu.SemaphoreType.DMA,
            pltpu.SemaphoreType.DMA,
        ],
    )
    return pl.pallas_call(
        _ffn_body,
        grid_spec=grid_spec,
        out_shape=jax.ShapeDtypeStruct((R, D), jnp.float32),
    )(tbl, xs, w_gate, w_up, w_down)


@functools.cache
def _get_scatter_x():
    mesh = plsc.VectorSubcoreMesh(core_axis_name="c", subcore_axis_name="s")
    return pl.kernel(
        _scatter_x_body,
        mesh=mesh,
        out_type=jax.ShapeDtypeStruct((R, D), jnp.float32),
        scratch_types=[
            pltpu.VMEM((SCH,), jnp.int32),
            pltpu.VMEM((SCH, D), jnp.float32),
            pltpu.SemaphoreType.DMA,
        ],
    )


def _scatter_x_body(x_hbm, p0_hbm, p1_hbm, xs_hbm, idx_v, rows_v, sem):
    k = lax.axis_index("c")
    t0 = lax.axis_index("s") * SCH

    @pl.when(k == 0)
    def _():
        pltpu.sync_copy(p0_hbm.at[pl.ds(t0, SCH)], idx_v)

    @pl.when(k == 1)
    def _():
        pltpu.sync_copy(p1_hbm.at[pl.ds(t0, SCH)], idx_v)

    pltpu.sync_copy(x_hbm.at[pl.ds(t0, SCH)], rows_v)
    pltpu.async_copy(rows_v, xs_hbm.at[idx_v], sem).wait()


@functools.cache
def _get_combine():
    mesh = plsc.VectorSubcoreMesh(core_axis_name="c", subcore_axis_name="s")
    return pl.kernel(
        _combine_body,
        mesh=mesh,
        out_type=jax.ShapeDtypeStruct((T, D), jnp.float32),
        scratch_types=[
            pltpu.VMEM((CCH,), jnp.int32),
            pltpu.VMEM((CCH,), jnp.int32),
            pltpu.VMEM((CCH,), jnp.int32),
            pltpu.VMEM((CCH,), jnp.int32),
            pltpu.VMEM((TPW,), jnp.float32),
            pltpu.VMEM((TPW,), jnp.float32),
            pltpu.VMEM((CCH, D), jnp.float32),
            pltpu.VMEM((CCH, D), jnp.float32),
            pltpu.VMEM((CCH, D), jnp.float32),
            pltpu.VMEM((CCH, D), jnp.float32),
            pltpu.SemaphoreType.DMA,
            pltpu.SemaphoreType.DMA,
        ],
    )


def _combine_body(y_hbm, p0_hbm, p1_hbm, w0_hbm, w1_hbm, out_hbm,
                  i0a_v, i1a_v, i0b_v, i1b_v, w0_v, w1_v,
                  r0a_v, r1a_v, r0b_v, r1b_v, sem_a, sem_b):
    wid = lax.axis_index("c") * _NS + lax.axis_index("s")
    tb = wid * TPW
    pltpu.sync_copy(w0_hbm.at[pl.ds(tb, TPW)], w0_v)
    pltpu.sync_copy(w1_hbm.at[pl.ds(tb, TPW)], w1_v)
    pltpu.sync_copy(p0_hbm.at[pl.ds(tb, CCH)], i0a_v)
    pltpu.sync_copy(p1_hbm.at[pl.ds(tb, CCH)], i1a_v)
    pltpu.sync_copy(p0_hbm.at[pl.ds(tb + CCH, CCH)], i0b_v)
    pltpu.sync_copy(p1_hbm.at[pl.ds(tb + CCH, CCH)], i1b_v)
    a0 = pltpu.async_copy(y_hbm.at[i0a_v], r0a_v, sem_a)
    a1 = pltpu.async_copy(y_hbm.at[i1a_v], r1a_v, sem_a)
    b0 = pltpu.async_copy(y_hbm.at[i0b_v], r0b_v, sem_b)
    b1 = pltpu.async_copy(y_hbm.at[i1b_v], r1b_v, sem_b)

    def weighted_sum(r0_v, r1_v, gbase):
        for g in range(CCH // _L):
            wv0 = w0_v[pl.ds((gbase + g) * _L, _L)]
            wv1 = w1_v[pl.ds((gbase + g) * _L, _L)]

            def body(lane, carry, wv0=wv0, wv1=wv1, g=g):
                iv = jnp.full((_L,), lane, jnp.int32)
                w0s = wv0.at[iv].get(mode="promise_in_bounds")
                w1s = wv1.at[iv].get(mode="promise_in_bounds")
                i = g * _L + lane
                for j in range(D // _L):
                    sl = pl.ds(j * _L, _L)
                    r0_v[i, sl] = w0s * r0_v[i, sl] + w1s * r1_v[i, sl]
                return carry

            lax.fori_loop(0, _L, body, 0, unroll=2)

    a0.wait()
    a1.wait()
    weighted_sum(r0a_v, r1a_v, 0)
    st_a = pltpu.async_copy(r0a_v, out_hbm.at[pl.ds(tb, CCH)], sem_a)
    b0.wait()
    b1.wait()
    weighted_sum(r0b_v, r1b_v, CCH // _L)
    st_a.wait()
    pltpu.sync_copy(r0b_v, out_hbm.at[pl.ds(tb + CCH, CCH)])


def kernel(hidden_states, gate_w, w_gate, w_up, w_down):
    b, s, d = hidden_states.shape
    x = hidden_states.reshape(-1, d)
    p0, p1, w0, w1, tbl = _route(x, gate_w)
    p0, p1 = p0.reshape(T), p1.reshape(T)
    w0, w1 = w0.reshape(T), w1.reshape(T)
    xs = _get_scatter_x()(x, p0, p1)
    ys = _ffn(tbl, xs, w_gate, w_up, w_down)
    out = _get_combine()(ys, p0, p1, w0, w1)
    return out.reshape(b, s, d)


# speedup vs baseline: 1.7337x; 1.0234x over previous
import functools

import jax
import jax.numpy as jnp
from jax import lax
from jax.experimental import pallas as pl
from jax.experimental.pallas import tpu as pltpu
from jax.experimental.pallas import tpu_sc as plsc

T = 2048
D = 768
F = 512
E = 16
TOPK = 2
BLK = 128
NB = 48
R = NB * BLK
P = T * TOPK

_NC, _NS, _L = 2, 16, 16
NW = _NC * _NS
SCH = T // _NS
TPW = T // NW
CCH = TPW // 2


def _route_body(x_ref, gw_ref, p0_ref, p1_ref, w0_ref, w1_ref, tbl_ref, xb_ref):
    x = x_ref[...]
    xu = lax.bitcast_convert_type(x, jnp.uint32)
    rnd = (xu + jnp.uint32(0x7FFF) +
           ((xu >> jnp.uint32(16)) & jnp.uint32(1))) >> jnp.uint32(16)
    lo16 = rnd[:, :D // 2]
    hi16 = rnd[:, D // 2:]
    xb_ref[...] = lax.bitcast_convert_type(
        lo16 | (hi16 << jnp.uint32(16)), jnp.int32)
    logits = jnp.dot(x, gw_ref[...], preferred_element_type=jnp.float32)
    lane = lax.broadcasted_iota(jnp.int32, (T, E), 1)
    m1 = jnp.max(logits, axis=1, keepdims=True)
    e1 = jnp.min(jnp.where(logits == m1, lane, E), axis=1, keepdims=True)
    masked = jnp.where(lane == e1, -jnp.inf, logits)
    m2 = jnp.max(masked, axis=1, keepdims=True)
    e2 = jnp.min(jnp.where(masked == m2, lane, E), axis=1, keepdims=True)
    w1 = jax.nn.sigmoid(m1 - m2)
    w0_ref[...] = w1
    w1_ref[...] = 1.0 - w1

    oh1 = (lane == e1).astype(jnp.float32)
    oh2 = (lane == e2).astype(jnp.float32)
    ri = lax.broadcasted_iota(jnp.int32, (T, T), 0)
    ci = lax.broadcasted_iota(jnp.int32, (T, T), 1)
    ltri = (ci <= ri).astype(jnp.float32)
    c1 = jnp.dot(ltri, oh1, preferred_element_type=jnp.float32)
    tot1 = jnp.sum(oh1, axis=0, keepdims=True)
    c2 = jnp.dot(ltri, oh2, preferred_element_type=jnp.float32) + tot1
    tot = tot1 + jnp.sum(oh2, axis=0, keepdims=True)
    nblk = jnp.floor((tot + (BLK - 1)) / BLK)
    si = lax.broadcasted_iota(jnp.int32, (E, E), 0)
    sj = lax.broadcasted_iota(jnp.int32, (E, E), 1)
    stri = (si < sj).astype(jnp.float32)
    bstart = jnp.dot(nblk, stri, preferred_element_type=jnp.float32)
    po = bstart * BLK
    rank0 = jnp.sum(oh1 * c1, axis=1, keepdims=True) - 1.0
    rank1 = jnp.sum(oh2 * c2, axis=1, keepdims=True) - 1.0
    po1 = jnp.sum(oh1 * po, axis=1, keepdims=True)
    po2 = jnp.sum(oh2 * po, axis=1, keepdims=True)
    p0_ref[...] = (po1 + rank0).astype(jnp.int32)
    p1_ref[...] = (po2 + rank1).astype(jnp.int32)
    tot = jnp.sum(nblk, keepdims=True)
    bif = lax.broadcasted_iota(jnp.int32, (NB, 1), 0).astype(jnp.float32)
    active = (nblk > 0.0)
    ind = (bif >= bstart).astype(jnp.float32)
    be = jnp.sum(ind, axis=1, keepdims=True) - 1.0
    chg = jnp.sum(((bstart == bif) & active).astype(jnp.float32), axis=1,
                  keepdims=True)
    runcnt = jnp.sum(((bstart <= bif) & active).astype(jnp.float32), axis=1,
                     keepdims=True)
    par = runcnt - 1.0 - 2.0 * jnp.floor((runcnt - 1.0) * 0.5)
    bigv = jnp.float32(1e9)
    startsf = jnp.where(active, bstart, bigv)
    cand = jnp.where(startsf > bif, startsf, bigv)
    nmin = jnp.min(cand, axis=1, keepdims=True)
    hn = (nmin < 1e8).astype(jnp.float32)
    eiota = lax.broadcasted_iota(jnp.int32, (NB, E), 1).astype(jnp.float32)
    nxt = jnp.sum(jnp.where(cand == nmin, eiota, 0.0), axis=1, keepdims=True)
    totb = jnp.zeros((NB, 1), jnp.float32) + tot
    tbl = jnp.concatenate([be, chg, par, nxt, hn, totb,
                           jnp.zeros((NB, 2), jnp.float32)], axis=1)
    tbl_ref[...] = tbl.astype(jnp.int32)


def _route(x, gate_w):
    return pl.pallas_call(
        _route_body,
        out_shape=(
            jax.ShapeDtypeStruct((T, 1), jnp.int32),
            jax.ShapeDtypeStruct((T, 1), jnp.int32),
            jax.ShapeDtypeStruct((T, 1), jnp.float32),
            jax.ShapeDtypeStruct((T, 1), jnp.float32),
            jax.ShapeDtypeStruct((NB, 8), jnp.int32),
            jax.ShapeDtypeStruct((T, D // 2), jnp.int32),
        ),
    )(x, gate_w)


def _ffn_body(tbl_ref, xs_ref, wg_ref, wu_ref, wd_ref, y_ref,
              xbuf, wgbuf, wubuf, wdbuf, ybuf, sx, sw, sy):
    tot = tbl_ref[0, 5]

    def wcopies(e, slot):
        return (pltpu.make_async_copy(wg_ref.at[e], wgbuf.at[slot], sw),
                pltpu.make_async_copy(wu_ref.at[e], wubuf.at[slot], sw),
                pltpu.make_async_copy(wd_ref.at[e], wdbuf.at[slot], sw))

    def xcopy(i, slot):
        return pltpu.make_async_copy(
            xs_ref.at[pl.ds(i * BLK, BLK)], xbuf.at[slot], sx)

    def ycopy(i, slot):
        return pltpu.make_async_copy(
            ybuf.at[slot], y_ref.at[pl.ds(i * BLK, BLK)], sy)

    xcopy(0, 0).start()
    for c in wcopies(tbl_ref[0, 0], 0):
        c.start()

    def step(i, carry):
        q = lax.rem(i, 2)
        bev = tbl_ref[i, 0]
        chg = tbl_ref[i, 1]
        par = tbl_ref[i, 2]
        nxt = tbl_ref[i, 3]
        hn = tbl_ref[i, 4]

        @pl.when(chg == 1)
        def _():
            for c in wcopies(bev, par):
                c.wait()

            @pl.when(hn == 1)
            def _():
                for c in wcopies(nxt, 1 - par):
                    c.start()

        xcopy(i, q).wait()

        @pl.when(i + 1 < tot)
        def _():
            xcopy(i + 1, 1 - q).start()

        @pl.when(i >= 2)
        def _():
            ycopy(i, q).wait()

        xw = xbuf[q]
        xlo = lax.bitcast_convert_type(xw << 16, jnp.float32)
        xhi = lax.bitcast_convert_type(xw & jnp.int32(-65536), jnp.float32)
        wg = wgbuf[par]
        wu = wubuf[par]
        g = (jnp.dot(xlo, wg[:D // 2], preferred_element_type=jnp.float32) +
             jnp.dot(xhi, wg[D // 2:], preferred_element_type=jnp.float32))
        u = (jnp.dot(xlo, wu[:D // 2], preferred_element_type=jnp.float32) +
             jnp.dot(xhi, wu[D // 2:], preferred_element_type=jnp.float32))
        h = g * jax.nn.sigmoid(g) * u
        ybuf[q, ...] = jnp.dot(h, wdbuf[par], preferred_element_type=jnp.float32)
        ycopy(i, q).start()
        return carry

    lax.fori_loop(0, tot, step, 0)
    ycopy(0, 0).wait()
    ycopy(0, 1).wait()


def _ffn(tbl, xs, w_gate, w_up, w_down):
    grid_spec = pltpu.PrefetchScalarGridSpec(
        num_scalar_prefetch=1,
        grid=(1,),
        in_specs=[
            pl.BlockSpec(memory_space=pl.ANY),
            pl.BlockSpec(memory_space=pl.ANY),
            pl.BlockSpec(memory_space=pl.ANY),
            pl.BlockSpec(memory_space=pl.ANY),
        ],
        out_specs=pl.BlockSpec(memory_space=pl.ANY),
        scratch_shapes=[
            pltpu.VMEM((2, BLK, D // 2), jnp.int32),
            pltpu.VMEM((2, D, F), jnp.float32),
            pltpu.VMEM((2, D, F), jnp.float32),
            pltpu.VMEM((2, F, D), jnp.float32),
            pltpu.VMEM((2, BLK, D), jnp.float32),
            pltpu.SemaphoreType.DMA,
            pltpu.SemaphoreType.DMA,
            pltpu.SemaphoreType.DMA,
        ],
    )
    return pl.pallas_call(
        _ffn_body,
        grid_spec=grid_spec,
        out_shape=jax.ShapeDtypeStruct((R, D), jnp.float32),
    )(tbl, xs, w_gate, w_up, w_down)


@functools.cache
def _get_scatter_x():
    mesh = plsc.VectorSubcoreMesh(core_axis_name="c", subcore_axis_name="s")
    return pl.kernel(
        _scatter_x_body,
        mesh=mesh,
        out_type=jax.ShapeDtypeStruct((R, D // 2), jnp.int32),
        scratch_types=[
            pltpu.VMEM((SCH,), jnp.int32),
            pltpu.VMEM((SCH, D // 2), jnp.int32),
            pltpu.SemaphoreType.DMA,
        ],
    )


def _scatter_x_body(x_hbm, p0_hbm, p1_hbm, xs_hbm, idx_v, rows_v, sem):
    k = lax.axis_index("c")
    t0 = lax.axis_index("s") * SCH

    @pl.when(k == 0)
    def _():
        pltpu.sync_copy(p0_hbm.at[pl.ds(t0, SCH)], idx_v)

    @pl.when(k == 1)
    def _():
        pltpu.sync_copy(p1_hbm.at[pl.ds(t0, SCH)], idx_v)

    pltpu.sync_copy(x_hbm.at[pl.ds(t0, SCH)], rows_v)
    pltpu.async_copy(rows_v, xs_hbm.at[idx_v], sem).wait()


@functools.cache
def _get_combine():
    mesh = plsc.VectorSubcoreMesh(core_axis_name="c", subcore_axis_name="s")
    return pl.kernel(
        _combine_body,
        mesh=mesh,
        out_type=jax.ShapeDtypeStruct((T, D), jnp.float32),
        scratch_types=[
            pltpu.VMEM((CCH,), jnp.int32),
            pltpu.VMEM((CCH,), jnp.int32),
            pltpu.VMEM((CCH,), jnp.int32),
            pltpu.VMEM((CCH,), jnp.int32),
            pltpu.VMEM((TPW,), jnp.float32),
            pltpu.VMEM((TPW,), jnp.float32),
            pltpu.VMEM((CCH, D), jnp.float32),
            pltpu.VMEM((CCH, D), jnp.float32),
            pltpu.VMEM((CCH, D), jnp.float32),
            pltpu.VMEM((CCH, D), jnp.float32),
            pltpu.SemaphoreType.DMA,
            pltpu.SemaphoreType.DMA,
        ],
    )


def _combine_body(y_hbm, p0_hbm, p1_hbm, w0_hbm, w1_hbm, out_hbm,
                  i0a_v, i1a_v, i0b_v, i1b_v, w0_v, w1_v,
                  r0a_v, r1a_v, r0b_v, r1b_v, sem_a, sem_b):
    wid = lax.axis_index("c") * _NS + lax.axis_index("s")
    tb = wid * TPW
    pltpu.sync_copy(w0_hbm.at[pl.ds(tb, TPW)], w0_v)
    pltpu.sync_copy(w1_hbm.at[pl.ds(tb, TPW)], w1_v)
    pltpu.sync_copy(p0_hbm.at[pl.ds(tb, CCH)], i0a_v)
    pltpu.sync_copy(p1_hbm.at[pl.ds(tb, CCH)], i1a_v)
    pltpu.sync_copy(p0_hbm.at[pl.ds(tb + CCH, CCH)], i0b_v)
    pltpu.sync_copy(p1_hbm.at[pl.ds(tb + CCH, CCH)], i1b_v)
    a0 = pltpu.async_copy(y_hbm.at[i0a_v], r0a_v, sem_a)
    a1 = pltpu.async_copy(y_hbm.at[i1a_v], r1a_v, sem_a)
    b0 = pltpu.async_copy(y_hbm.at[i0b_v], r0b_v, sem_b)
    b1 = pltpu.async_copy(y_hbm.at[i1b_v], r1b_v, sem_b)

    def weighted_sum(r0_v, r1_v, gbase):
        for g in range(CCH // _L):
            wv0 = w0_v[pl.ds((gbase + g) * _L, _L)]
            wv1 = w1_v[pl.ds((gbase + g) * _L, _L)]

            def body(lane, carry, wv0=wv0, wv1=wv1, g=g):
                iv = jnp.full((_L,), lane, jnp.int32)
                w0s = wv0.at[iv].get(mode="promise_in_bounds")
                w1s = wv1.at[iv].get(mode="promise_in_bounds")
                i = g * _L + lane
                for j in range(D // _L):
                    sl = pl.ds(j * _L, _L)
                    r0_v[i, sl] = w0s * r0_v[i, sl] + w1s * r1_v[i, sl]
                return carry

            lax.fori_loop(0, _L, body, 0, unroll=2)

    a0.wait()
    a1.wait()
    weighted_sum(r0a_v, r1a_v, 0)
    st_a = pltpu.async_copy(r0a_v, out_hbm.at[pl.ds(tb, CCH)], sem_a)
    b0.wait()
    b1.wait()
    weighted_sum(r0b_v, r1b_v, CCH // _L)
    st_a.wait()
    pltpu.sync_copy(r0b_v, out_hbm.at[pl.ds(tb + CCH, CCH)])


def kernel(hidden_states, gate_w, w_gate, w_up, w_down):
    b, s, d = hidden_states.shape
    x = hidden_states.reshape(-1, d)
    p0, p1, w0, w1, tbl, xb = _route(x, gate_w)
    p0, p1 = p0.reshape(T), p1.reshape(T)
    w0, w1 = w0.reshape(T), w1.reshape(T)
    xs = _get_scatter_x()(xb, p0, p1)
    ys = _ffn(tbl, xs, w_gate, w_up, w_down)
    out = _get_combine()(ys, p0, p1, w0, w1)
    return out.reshape(b, s, d)


# speedup vs baseline: 1.7529x; 1.0111x over previous
import functools

import jax
import jax.numpy as jnp
from jax import lax
from jax.experimental import pallas as pl
from jax.experimental.pallas import tpu as pltpu
from jax.experimental.pallas import tpu_sc as plsc

T = 2048
D = 768
F = 512
E = 16
TOPK = 2
BLK = 128
NB = 48
R = NB * BLK
P = T * TOPK

_NC, _NS, _L = 2, 16, 16
NW = _NC * _NS
SCH = T // _NS
TPW = T // NW
CCH = TPW // 2


def _route_body(x_ref, gw_ref, p0_ref, p1_ref, w0_ref, w1_ref, tbl_ref, xb_ref):
    x = x_ref[...]
    xu = lax.bitcast_convert_type(x, jnp.uint32)
    rnd = (xu + jnp.uint32(0x7FFF) +
           ((xu >> jnp.uint32(16)) & jnp.uint32(1))) >> jnp.uint32(16)
    lo16 = rnd[:, :D // 2]
    hi16 = rnd[:, D // 2:]
    xb_ref[...] = lax.bitcast_convert_type(
        lo16 | (hi16 << jnp.uint32(16)), jnp.int32)
    logits = jnp.dot(x, gw_ref[...], preferred_element_type=jnp.float32)
    lane = lax.broadcasted_iota(jnp.int32, (T, E), 1)
    m1 = jnp.max(logits, axis=1, keepdims=True)
    e1 = jnp.min(jnp.where(logits == m1, lane, E), axis=1, keepdims=True)
    masked = jnp.where(lane == e1, -jnp.inf, logits)
    m2 = jnp.max(masked, axis=1, keepdims=True)
    e2 = jnp.min(jnp.where(masked == m2, lane, E), axis=1, keepdims=True)
    w1 = jax.nn.sigmoid(m1 - m2)
    w0_ref[...] = w1
    w1_ref[...] = 1.0 - w1

    oh1 = (lane == e1).astype(jnp.float32)
    oh2 = (lane == e2).astype(jnp.float32)
    ri = lax.broadcasted_iota(jnp.int32, (T, T), 0)
    ci = lax.broadcasted_iota(jnp.int32, (T, T), 1)
    ltri = (ci <= ri).astype(jnp.float32)
    c1 = jnp.dot(ltri, oh1, preferred_element_type=jnp.float32)
    tot1 = jnp.sum(oh1, axis=0, keepdims=True)
    c2 = jnp.dot(ltri, oh2, preferred_element_type=jnp.float32) + tot1
    tot = tot1 + jnp.sum(oh2, axis=0, keepdims=True)
    nblk = jnp.floor((tot + (BLK - 1)) / BLK)
    si = lax.broadcasted_iota(jnp.int32, (E, E), 0)
    sj = lax.broadcasted_iota(jnp.int32, (E, E), 1)
    stri = (si < sj).astype(jnp.float32)
    bstart = jnp.dot(nblk, stri, preferred_element_type=jnp.float32)
    po = bstart * BLK
    rank0 = jnp.sum(oh1 * c1, axis=1, keepdims=True) - 1.0
    rank1 = jnp.sum(oh2 * c2, axis=1, keepdims=True) - 1.0
    po1 = jnp.sum(oh1 * po, axis=1, keepdims=True)
    po2 = jnp.sum(oh2 * po, axis=1, keepdims=True)
    p0_ref[...] = (po1 + rank0).astype(jnp.int32)
    p1_ref[...] = (po2 + rank1).astype(jnp.int32)
    tot = jnp.sum(nblk, keepdims=True)
    bif = lax.broadcasted_iota(jnp.int32, (NB, 1), 0).astype(jnp.float32)
    active = (nblk > 0.0)
    ind = (bif >= bstart).astype(jnp.float32)
    be = jnp.sum(ind, axis=1, keepdims=True) - 1.0
    chg = jnp.sum(((bstart == bif) & active).astype(jnp.float32), axis=1,
                  keepdims=True)
    runcnt = jnp.sum(((bstart <= bif) & active).astype(jnp.float32), axis=1,
                     keepdims=True)
    runm1 = runcnt - 1.0
    par = runm1 - 3.0 * jnp.floor(runm1 / 3.0)
    bigv = jnp.float32(1e9)
    startsf = jnp.where(active, bstart, bigv)
    cand = jnp.where(startsf > bif, startsf, bigv)
    nmin = jnp.min(cand, axis=1, keepdims=True)
    hn = (nmin < 1e8).astype(jnp.float32)
    eiota = lax.broadcasted_iota(jnp.int32, (NB, E), 1).astype(jnp.float32)
    nxt = jnp.sum(jnp.where(cand == nmin, eiota, 0.0), axis=1, keepdims=True)
    cand2 = jnp.where(cand == nmin, bigv, cand)
    n2min = jnp.min(cand2, axis=1, keepdims=True)
    hn2 = (n2min < 1e8).astype(jnp.float32)
    nxt2 = jnp.sum(jnp.where(cand2 == n2min, eiota, 0.0), axis=1, keepdims=True)
    totb = jnp.zeros((NB, 1), jnp.float32) + tot
    tbl = jnp.concatenate([be, chg, par, nxt, hn, totb, nxt2, hn2], axis=1)
    tbl_ref[...] = tbl.astype(jnp.int32)


def _route(x, gate_w):
    return pl.pallas_call(
        _route_body,
        out_shape=(
            jax.ShapeDtypeStruct((T, 1), jnp.int32),
            jax.ShapeDtypeStruct((T, 1), jnp.int32),
            jax.ShapeDtypeStruct((T, 1), jnp.float32),
            jax.ShapeDtypeStruct((T, 1), jnp.float32),
            jax.ShapeDtypeStruct((NB, 8), jnp.int32),
            jax.ShapeDtypeStruct((T, D // 2), jnp.int32),
        ),
    )(x, gate_w)


def _ffn_body(tbl_ref, xs_ref, wg_ref, wu_ref, wd_ref, y_ref,
              xbuf, wgbuf, wubuf, wdbuf, ybuf, sx, sw, sy):
    tot = tbl_ref[0, 5]

    def wcopies(e, slot):
        h1, h2 = pl.ds(0, D // 2), pl.ds(D // 2, D // 2)
        f1, f2 = pl.ds(0, F // 2), pl.ds(F // 2, F // 2)
        return (pltpu.make_async_copy(wg_ref.at[e, h1], wgbuf.at[slot, h1], sw),
                pltpu.make_async_copy(wg_ref.at[e, h2], wgbuf.at[slot, h2], sw),
                pltpu.make_async_copy(wu_ref.at[e, h1], wubuf.at[slot, h1], sw),
                pltpu.make_async_copy(wu_ref.at[e, h2], wubuf.at[slot, h2], sw),
                pltpu.make_async_copy(wd_ref.at[e, f1], wdbuf.at[slot, f1], sw),
                pltpu.make_async_copy(wd_ref.at[e, f2], wdbuf.at[slot, f2], sw))

    def xcopy(i, slot):
        return pltpu.make_async_copy(
            xs_ref.at[pl.ds(i * BLK, BLK)], xbuf.at[slot], sx)

    def ycopy(i, slot):
        return pltpu.make_async_copy(
            ybuf.at[slot], y_ref.at[pl.ds(i * BLK, BLK)], sy)

    xcopy(0, 0).start()
    for c in wcopies(tbl_ref[0, 0], 0):
        c.start()

    @pl.when(tbl_ref[0, 4] == 1)
    def _():
        for c in wcopies(tbl_ref[0, 3], 1):
            c.start()

    def step(i, carry):
        q = lax.rem(i, 2)
        bev = tbl_ref[i, 0]
        chg = tbl_ref[i, 1]
        par = tbl_ref[i, 2]
        nxt = tbl_ref[i, 3]
        hn = tbl_ref[i, 4]

        hn2 = tbl_ref[i, 7]
        nxt2 = tbl_ref[i, 6]

        @pl.when(chg == 1)
        def _():
            for c in wcopies(bev, par):
                c.wait()

            @pl.when(hn2 == 1)
            def _():
                slot2 = jnp.where(par >= 1, par - 1, 2)
                for c in wcopies(nxt2, slot2):
                    c.start()

        xcopy(i, q).wait()

        @pl.when(i + 1 < tot)
        def _():
            xcopy(i + 1, 1 - q).start()

        @pl.when(i >= 2)
        def _():
            ycopy(i, q).wait()

        xw = xbuf[q]
        xlo = lax.bitcast_convert_type(xw << 16, jnp.float32)
        xhi = lax.bitcast_convert_type(xw & jnp.int32(-65536), jnp.float32)
        wg = wgbuf[par]
        wu = wubuf[par]
        g = (jnp.dot(xlo, wg[:D // 2], preferred_element_type=jnp.float32) +
             jnp.dot(xhi, wg[D // 2:], preferred_element_type=jnp.float32))
        u = (jnp.dot(xlo, wu[:D // 2], preferred_element_type=jnp.float32) +
             jnp.dot(xhi, wu[D // 2:], preferred_element_type=jnp.float32))
        h = g * jax.nn.sigmoid(g) * u
        ybuf[q, ...] = jnp.dot(h, wdbuf[par], preferred_element_type=jnp.float32)
        ycopy(i, q).start()
        return carry

    lax.fori_loop(0, tot, step, 0)
    ycopy(0, 0).wait()
    ycopy(0, 1).wait()


def _ffn(tbl, xs, w_gate, w_up, w_down):
    grid_spec = pltpu.PrefetchScalarGridSpec(
        num_scalar_prefetch=1,
        grid=(1,),
        in_specs=[
            pl.BlockSpec(memory_space=pl.ANY),
            pl.BlockSpec(memory_space=pl.ANY),
            pl.BlockSpec(memory_space=pl.ANY),
            pl.BlockSpec(memory_space=pl.ANY),
        ],
        out_specs=pl.BlockSpec(memory_space=pl.ANY),
        scratch_shapes=[
            pltpu.VMEM((2, BLK, D // 2), jnp.int32),
            pltpu.VMEM((3, D, F), jnp.float32),
            pltpu.VMEM((3, D, F), jnp.float32),
            pltpu.VMEM((3, F, D), jnp.float32),
            pltpu.VMEM((2, BLK, D), jnp.float32),
            pltpu.SemaphoreType.DMA,
            pltpu.SemaphoreType.DMA,
            pltpu.SemaphoreType.DMA,
        ],
    )
    return pl.pallas_call(
        _ffn_body,
        grid_spec=grid_spec,
        out_shape=jax.ShapeDtypeStruct((R, D), jnp.float32),
    )(tbl, xs, w_gate, w_up, w_down)


@functools.cache
def _get_scatter_x():
    mesh = plsc.VectorSubcoreMesh(core_axis_name="c", subcore_axis_name="s")
    return pl.kernel(
        _scatter_x_body,
        mesh=mesh,
        out_type=jax.ShapeDtypeStruct((R, D // 2), jnp.int32),
        scratch_types=[
            pltpu.VMEM((SCH,), jnp.int32),
            pltpu.VMEM((SCH, D // 2), jnp.int32),
            pltpu.SemaphoreType.DMA,
        ],
    )


def _scatter_x_body(x_hbm, p0_hbm, p1_hbm, xs_hbm, idx_v, rows_v, sem):
    k = lax.axis_index("c")
    t0 = lax.axis_index("s") * SCH

    @pl.when(k == 0)
    def _():
        pltpu.sync_copy(p0_hbm.at[pl.ds(t0, SCH)], idx_v)

    @pl.when(k == 1)
    def _():
        pltpu.sync_copy(p1_hbm.at[pl.ds(t0, SCH)], idx_v)

    pltpu.sync_copy(x_hbm.at[pl.ds(t0, SCH)], rows_v)
    pltpu.async_copy(rows_v, xs_hbm.at[idx_v], sem).wait()


@functools.cache
def _get_combine():
    mesh = plsc.VectorSubcoreMesh(core_axis_name="c", subcore_axis_name="s")
    return pl.kernel(
        _combine_body,
        mesh=mesh,
        out_type=jax.ShapeDtypeStruct((T, D), jnp.float32),
        scratch_types=[
            pltpu.VMEM((CCH,), jnp.int32),
            pltpu.VMEM((CCH,), jnp.int32),
            pltpu.VMEM((CCH,), jnp.int32),
            pltpu.VMEM((CCH,), jnp.int32),
            pltpu.VMEM((TPW,), jnp.float32),
            pltpu.VMEM((TPW,), jnp.float32),
            pltpu.VMEM((CCH, D), jnp.float32),
            pltpu.VMEM((CCH, D), jnp.float32),
            pltpu.VMEM((CCH, D), jnp.float32),
            pltpu.VMEM((CCH, D), jnp.float32),
            pltpu.SemaphoreType.DMA,
            pltpu.SemaphoreType.DMA,
        ],
    )


def _combine_body(y_hbm, p0_hbm, p1_hbm, w0_hbm, w1_hbm, out_hbm,
                  i0a_v, i1a_v, i0b_v, i1b_v, w0_v, w1_v,
                  r0a_v, r1a_v, r0b_v, r1b_v, sem_a, sem_b):
    wid = lax.axis_index("c") * _NS + lax.axis_index("s")
    tb = wid * TPW
    pltpu.sync_copy(w0_hbm.at[pl.ds(tb, TPW)], w0_v)
    pltpu.sync_copy(w1_hbm.at[pl.ds(tb, TPW)], w1_v)
    pltpu.sync_copy(p0_hbm.at[pl.ds(tb, CCH)], i0a_v)
    pltpu.sync_copy(p1_hbm.at[pl.ds(tb, CCH)], i1a_v)
    pltpu.sync_copy(p0_hbm.at[pl.ds(tb + CCH, CCH)], i0b_v)
    pltpu.sync_copy(p1_hbm.at[pl.ds(tb + CCH, CCH)], i1b_v)
    a0 = pltpu.async_copy(y_hbm.at[i0a_v], r0a_v, sem_a)
    a1 = pltpu.async_copy(y_hbm.at[i1a_v], r1a_v, sem_a)
    b0 = pltpu.async_copy(y_hbm.at[i0b_v], r0b_v, sem_b)
    b1 = pltpu.async_copy(y_hbm.at[i1b_v], r1b_v, sem_b)

    def weighted_sum(r0_v, r1_v, gbase):
        for g in range(CCH // _L):
            wv0 = w0_v[pl.ds((gbase + g) * _L, _L)]
            wv1 = w1_v[pl.ds((gbase + g) * _L, _L)]

            def body(lane, carry, wv0=wv0, wv1=wv1, g=g):
                iv = jnp.full((_L,), lane, jnp.int32)
                w0s = wv0.at[iv].get(mode="promise_in_bounds")
                w1s = wv1.at[iv].get(mode="promise_in_bounds")
                i = g * _L + lane
                for j in range(D // _L):
                    sl = pl.ds(j * _L, _L)
                    r0_v[i, sl] = w0s * r0_v[i, sl] + w1s * r1_v[i, sl]
                return carry

            lax.fori_loop(0, _L, body, 0, unroll=2)

    a0.wait()
    a1.wait()
    weighted_sum(r0a_v, r1a_v, 0)
    st_a = pltpu.async_copy(r0a_v, out_hbm.at[pl.ds(tb, CCH)], sem_a)
    b0.wait()
    b1.wait()
    weighted_sum(r0b_v, r1b_v, CCH // _L)
    st_a.wait()
    pltpu.sync_copy(r0b_v, out_hbm.at[pl.ds(tb + CCH, CCH)])


def kernel(hidden_states, gate_w, w_gate, w_up, w_down):
    b, s, d = hidden_states.shape
    x = hidden_states.reshape(-1, d)
    p0, p1, w0, w1, tbl, xb = _route(x, gate_w)
    p0, p1 = p0.reshape(T), p1.reshape(T)
    w0, w1 = w0.reshape(T), w1.reshape(T)
    xs = _get_scatter_x()(xb, p0, p1)
    ys = _ffn(tbl, xs, w_gate, w_up, w_down)
    out = _get_combine()(ys, p0, p1, w0, w1)
    return out.reshape(b, s, d)


# speedup vs baseline: 1.9940x; 1.1376x over previous
import functools

import jax
import jax.numpy as jnp
from jax import lax
from jax.experimental import pallas as pl
from jax.experimental.pallas import tpu as pltpu
from jax.experimental.pallas import tpu_sc as plsc

T = 2048
D = 768
F = 512
E = 16
TOPK = 2
BLK = 256
NB = 31
R = NB * BLK
P = T * TOPK

_NC, _NS, _L = 2, 16, 16
NW = _NC * _NS
SCH = T // _NS
TPW = T // NW
CCH = TPW // 2


def _route_body(x_ref, gw_ref, p0_ref, p1_ref, w0_ref, w1_ref, tbl_ref, xb_ref):
    x = x_ref[...]
    xu = lax.bitcast_convert_type(x, jnp.uint32)
    rnd = (xu + jnp.uint32(0x7FFF) +
           ((xu >> jnp.uint32(16)) & jnp.uint32(1))) >> jnp.uint32(16)
    lo16 = rnd[:, :D // 2]
    hi16 = rnd[:, D // 2:]
    xb_ref[...] = lax.bitcast_convert_type(
        lo16 | (hi16 << jnp.uint32(16)), jnp.int32)
    logits = jnp.dot(x, gw_ref[...], preferred_element_type=jnp.float32)
    lane = lax.broadcasted_iota(jnp.int32, (T, E), 1)
    m1 = jnp.max(logits, axis=1, keepdims=True)
    e1 = jnp.min(jnp.where(logits == m1, lane, E), axis=1, keepdims=True)
    masked = jnp.where(lane == e1, -jnp.inf, logits)
    m2 = jnp.max(masked, axis=1, keepdims=True)
    e2 = jnp.min(jnp.where(masked == m2, lane, E), axis=1, keepdims=True)
    w1 = jax.nn.sigmoid(m1 - m2)
    w0_ref[...] = w1
    w1_ref[...] = 1.0 - w1

    oh1 = (lane == e1).astype(jnp.float32)
    oh2 = (lane == e2).astype(jnp.float32)
    ri = lax.broadcasted_iota(jnp.int32, (T, T), 0)
    ci = lax.broadcasted_iota(jnp.int32, (T, T), 1)
    ltri = (ci <= ri).astype(jnp.float32)
    c1 = jnp.dot(ltri, oh1, preferred_element_type=jnp.float32)
    tot1 = jnp.sum(oh1, axis=0, keepdims=True)
    c2 = jnp.dot(ltri, oh2, preferred_element_type=jnp.float32) + tot1
    tot = tot1 + jnp.sum(oh2, axis=0, keepdims=True)
    nblk = jnp.floor((tot + (BLK - 1)) / BLK)
    si = lax.broadcasted_iota(jnp.int32, (E, E), 0)
    sj = lax.broadcasted_iota(jnp.int32, (E, E), 1)
    stri = (si < sj).astype(jnp.float32)
    bstart = jnp.dot(nblk, stri, preferred_element_type=jnp.float32)
    po = bstart * BLK
    rank0 = jnp.sum(oh1 * c1, axis=1, keepdims=True) - 1.0
    rank1 = jnp.sum(oh2 * c2, axis=1, keepdims=True) - 1.0
    po1 = jnp.sum(oh1 * po, axis=1, keepdims=True)
    po2 = jnp.sum(oh2 * po, axis=1, keepdims=True)
    p0_ref[...] = (po1 + rank0).astype(jnp.int32)
    p1_ref[...] = (po2 + rank1).astype(jnp.int32)
    tot = jnp.sum(nblk, keepdims=True)
    bif = lax.broadcasted_iota(jnp.int32, (NB, 1), 0).astype(jnp.float32)
    active = (nblk > 0.0)
    ind = (bif >= bstart).astype(jnp.float32)
    be = jnp.sum(ind, axis=1, keepdims=True) - 1.0
    chg = jnp.sum(((bstart == bif) & active).astype(jnp.float32), axis=1,
                  keepdims=True)
    runcnt = jnp.sum(((bstart <= bif) & active).astype(jnp.float32), axis=1,
                     keepdims=True)
    runm1 = runcnt - 1.0
    par = runm1 - 3.0 * jnp.floor(runm1 / 3.0)
    bigv = jnp.float32(1e9)
    startsf = jnp.where(active, bstart, bigv)
    cand = jnp.where(startsf > bif, startsf, bigv)
    nmin = jnp.min(cand, axis=1, keepdims=True)
    hn = (nmin < 1e8).astype(jnp.float32)
    eiota = lax.broadcasted_iota(jnp.int32, (NB, E), 1).astype(jnp.float32)
    nxt = jnp.sum(jnp.where(cand == nmin, eiota, 0.0), axis=1, keepdims=True)
    cand2 = jnp.where(cand == nmin, bigv, cand)
    n2min = jnp.min(cand2, axis=1, keepdims=True)
    hn2 = (n2min < 1e8).astype(jnp.float32)
    nxt2 = jnp.sum(jnp.where(cand2 == n2min, eiota, 0.0), axis=1, keepdims=True)
    totb = jnp.zeros((NB, 1), jnp.float32) + tot
    tbl = jnp.concatenate([be, chg, par, nxt, hn, totb, nxt2, hn2], axis=1)
    tbl_ref[...] = tbl.astype(jnp.int32)


def _route(x, gate_w):
    return pl.pallas_call(
        _route_body,
        out_shape=(
            jax.ShapeDtypeStruct((T, 1), jnp.int32),
            jax.ShapeDtypeStruct((T, 1), jnp.int32),
            jax.ShapeDtypeStruct((T, 1), jnp.float32),
            jax.ShapeDtypeStruct((T, 1), jnp.float32),
            jax.ShapeDtypeStruct((NB, 8), jnp.int32),
            jax.ShapeDtypeStruct((T, D // 2), jnp.int32),
        ),
    )(x, gate_w)


def _ffn_body(tbl_ref, xs_ref, wg_ref, wu_ref, wd_ref, y_ref,
              xbuf, wgbuf, wubuf, wdbuf, ybuf, sx, sw, sy):
    tot = tbl_ref[0, 5]

    def wcopies(e, slot):
        h1, h2 = pl.ds(0, D // 2), pl.ds(D // 2, D // 2)
        f1, f2 = pl.ds(0, F // 2), pl.ds(F // 2, F // 2)
        return (pltpu.make_async_copy(wg_ref.at[e, h1], wgbuf.at[slot, h1], sw),
                pltpu.make_async_copy(wg_ref.at[e, h2], wgbuf.at[slot, h2], sw),
                pltpu.make_async_copy(wu_ref.at[e, h1], wubuf.at[slot, h1], sw),
                pltpu.make_async_copy(wu_ref.at[e, h2], wubuf.at[slot, h2], sw),
                pltpu.make_async_copy(wd_ref.at[e, f1], wdbuf.at[slot, f1], sw),
                pltpu.make_async_copy(wd_ref.at[e, f2], wdbuf.at[slot, f2], sw))

    def xcopy(i, slot):
        return pltpu.make_async_copy(
            xs_ref.at[pl.ds(i * BLK, BLK)], xbuf.at[slot], sx)

    def ycopy(i, slot):
        return pltpu.make_async_copy(
            ybuf.at[slot], y_ref.at[pl.ds(i * BLK, BLK)], sy)

    xcopy(0, 0).start()
    for c in wcopies(tbl_ref[0, 0], 0):
        c.start()

    @pl.when(tbl_ref[0, 4] == 1)
    def _():
        for c in wcopies(tbl_ref[0, 3], 1):
            c.start()

    def step(i, carry):
        q = lax.rem(i, 2)
        bev = tbl_ref[i, 0]
        chg = tbl_ref[i, 1]
        par = tbl_ref[i, 2]
        nxt = tbl_ref[i, 3]
        hn = tbl_ref[i, 4]

        hn2 = tbl_ref[i, 7]
        nxt2 = tbl_ref[i, 6]

        @pl.when(chg == 1)
        def _():
            for c in wcopies(bev, par):
                c.wait()

            @pl.when(hn2 == 1)
            def _():
                slot2 = jnp.where(par >= 1, par - 1, 2)
                for c in wcopies(nxt2, slot2):
                    c.start()

        xcopy(i, q).wait()

        @pl.when(i + 1 < tot)
        def _():
            xcopy(i + 1, 1 - q).start()

        @pl.when(i >= 2)
        def _():
            ycopy(i, q).wait()

        xw = xbuf[q]
        xlo = lax.bitcast_convert_type(xw << 16, jnp.float32)
        xhi = lax.bitcast_convert_type(xw & jnp.int32(-65536), jnp.float32)
        wg = wgbuf[par]
        wu = wubuf[par]
        g = (jnp.dot(xlo, wg[:D // 2], preferred_element_type=jnp.float32) +
             jnp.dot(xhi, wg[D // 2:], preferred_element_type=jnp.float32))
        u = (jnp.dot(xlo, wu[:D // 2], preferred_element_type=jnp.float32) +
             jnp.dot(xhi, wu[D // 2:], preferred_element_type=jnp.float32))
        h = g * jax.nn.sigmoid(g) * u
        ybuf[q, ...] = jnp.dot(h, wdbuf[par], preferred_element_type=jnp.float32)
        ycopy(i, q).start()
        return carry

    lax.fori_loop(0, tot, step, 0)
    ycopy(0, 0).wait()
    ycopy(0, 1).wait()


def _ffn(tbl, xs, w_gate, w_up, w_down):
    grid_spec = pltpu.PrefetchScalarGridSpec(
        num_scalar_prefetch=1,
        grid=(1,),
        in_specs=[
            pl.BlockSpec(memory_space=pl.ANY),
            pl.BlockSpec(memory_space=pl.ANY),
            pl.BlockSpec(memory_space=pl.ANY),
            pl.BlockSpec(memory_space=pl.ANY),
        ],
        out_specs=pl.BlockSpec(memory_space=pl.ANY),
        scratch_shapes=[
            pltpu.VMEM((2, BLK, D // 2), jnp.int32),
            pltpu.VMEM((3, D, F), jnp.float32),
            pltpu.VMEM((3, D, F), jnp.float32),
            pltpu.VMEM((3, F, D), jnp.float32),
            pltpu.VMEM((2, BLK, D), jnp.float32),
            pltpu.SemaphoreType.DMA,
            pltpu.SemaphoreType.DMA,
            pltpu.SemaphoreType.DMA,
        ],
    )
    return pl.pallas_call(
        _ffn_body,
        grid_spec=grid_spec,
        out_shape=jax.ShapeDtypeStruct((R, D), jnp.float32),
    )(tbl, xs, w_gate, w_up, w_down)


@functools.cache
def _get_scatter_x():
    mesh = plsc.VectorSubcoreMesh(core_axis_name="c", subcore_axis_name="s")
    return pl.kernel(
        _scatter_x_body,
        mesh=mesh,
        out_type=jax.ShapeDtypeStruct((R, D // 2), jnp.int32),
        scratch_types=[
            pltpu.VMEM((SCH,), jnp.int32),
            pltpu.VMEM((SCH, D // 2), jnp.int32),
            pltpu.SemaphoreType.DMA,
        ],
    )


def _scatter_x_body(x_hbm, p0_hbm, p1_hbm, xs_hbm, idx_v, rows_v, sem):
    k = lax.axis_index("c")
    t0 = lax.axis_index("s") * SCH

    @pl.when(k == 0)
    def _():
        pltpu.sync_copy(p0_hbm.at[pl.ds(t0, SCH)], idx_v)

    @pl.when(k == 1)
    def _():
        pltpu.sync_copy(p1_hbm.at[pl.ds(t0, SCH)], idx_v)

    pltpu.sync_copy(x_hbm.at[pl.ds(t0, SCH)], rows_v)
    pltpu.async_copy(rows_v, xs_hbm.at[idx_v], sem).wait()


@functools.cache
def _get_combine():
    mesh = plsc.VectorSubcoreMesh(core_axis_name="c", subcore_axis_name="s")
    return pl.kernel(
        _combine_body,
        mesh=mesh,
        out_type=jax.ShapeDtypeStruct((T, D), jnp.float32),
        scratch_types=[
            pltpu.VMEM((CCH,), jnp.int32),
            pltpu.VMEM((CCH,), jnp.int32),
            pltpu.VMEM((CCH,), jnp.int32),
            pltpu.VMEM((CCH,), jnp.int32),
            pltpu.VMEM((TPW,), jnp.float32),
            pltpu.VMEM((TPW,), jnp.float32),
            pltpu.VMEM((CCH, D), jnp.float32),
            pltpu.VMEM((CCH, D), jnp.float32),
            pltpu.VMEM((CCH, D), jnp.float32),
            pltpu.VMEM((CCH, D), jnp.float32),
            pltpu.SemaphoreType.DMA,
            pltpu.SemaphoreType.DMA,
        ],
    )


def _combine_body(y_hbm, p0_hbm, p1_hbm, w0_hbm, w1_hbm, out_hbm,
                  i0a_v, i1a_v, i0b_v, i1b_v, w0_v, w1_v,
                  r0a_v, r1a_v, r0b_v, r1b_v, sem_a, sem_b):
    wid = lax.axis_index("c") * _NS + lax.axis_index("s")
    tb = wid * TPW
    pltpu.sync_copy(w0_hbm.at[pl.ds(tb, TPW)], w0_v)
    pltpu.sync_copy(w1_hbm.at[pl.ds(tb, TPW)], w1_v)
    pltpu.sync_copy(p0_hbm.at[pl.ds(tb, CCH)], i0a_v)
    pltpu.sync_copy(p1_hbm.at[pl.ds(tb, CCH)], i1a_v)
    pltpu.sync_copy(p0_hbm.at[pl.ds(tb + CCH, CCH)], i0b_v)
    pltpu.sync_copy(p1_hbm.at[pl.ds(tb + CCH, CCH)], i1b_v)
    a0 = pltpu.async_copy(y_hbm.at[i0a_v], r0a_v, sem_a)
    a1 = pltpu.async_copy(y_hbm.at[i1a_v], r1a_v, sem_a)
    b0 = pltpu.async_copy(y_hbm.at[i0b_v], r0b_v, sem_b)
    b1 = pltpu.async_copy(y_hbm.at[i1b_v], r1b_v, sem_b)

    def weighted_sum(r0_v, r1_v, gbase):
        for g in range(CCH // _L):
            wv0 = w0_v[pl.ds((gbase + g) * _L, _L)]
            wv1 = w1_v[pl.ds((gbase + g) * _L, _L)]

            def body(lane, carry, wv0=wv0, wv1=wv1, g=g):
                iv = jnp.full((_L,), lane, jnp.int32)
                w0s = wv0.at[iv].get(mode="promise_in_bounds")
                w1s = wv1.at[iv].get(mode="promise_in_bounds")
                i = g * _L + lane
                for j in range(D // _L):
                    sl = pl.ds(j * _L, _L)
                    r0_v[i, sl] = w0s * r0_v[i, sl] + w1s * r1_v[i, sl]
                return carry

            lax.fori_loop(0, _L, body, 0, unroll=2)

    a0.wait()
    a1.wait()
    weighted_sum(r0a_v, r1a_v, 0)
    st_a = pltpu.async_copy(r0a_v, out_hbm.at[pl.ds(tb, CCH)], sem_a)
    b0.wait()
    b1.wait()
    weighted_sum(r0b_v, r1b_v, CCH // _L)
    st_a.wait()
    pltpu.sync_copy(r0b_v, out_hbm.at[pl.ds(tb + CCH, CCH)])


def kernel(hidden_states, gate_w, w_gate, w_up, w_down):
    b, s, d = hidden_states.shape
    x = hidden_states.reshape(-1, d)
    p0, p1, w0, w1, tbl, xb = _route(x, gate_w)
    p0, p1 = p0.reshape(T), p1.reshape(T)
    w0, w1 = w0.reshape(T), w1.reshape(T)
    xs = _get_scatter_x()(xb, p0, p1)
    ys = _ffn(tbl, xs, w_gate, w_up, w_down)
    out = _get_combine()(ys, p0, p1, w0, w1)
    return out.reshape(b, s, d)


# speedup vs baseline: 2.0797x; 1.0430x over previous
import functools

import jax
import jax.numpy as jnp
from jax import lax
from jax.experimental import pallas as pl
from jax.experimental.pallas import tpu as pltpu
from jax.experimental.pallas import tpu_sc as plsc

T = 2048
D = 768
F = 512
E = 16
TOPK = 2
BLK = 512
NB = 23
R = NB * BLK
P = T * TOPK

_NC, _NS, _L = 2, 16, 16
NW = _NC * _NS
SCH = T // _NS
TPW = T // NW
CCH = TPW // 2


def _route_body(x_ref, gw_ref, p0_ref, p1_ref, w0_ref, w1_ref, tbl_ref, xb_ref):
    x = x_ref[...]
    xu = lax.bitcast_convert_type(x, jnp.uint32)
    rnd = (xu + jnp.uint32(0x7FFF) +
           ((xu >> jnp.uint32(16)) & jnp.uint32(1))) >> jnp.uint32(16)
    lo16 = rnd[:, :D // 2]
    hi16 = rnd[:, D // 2:]
    xb_ref[...] = lax.bitcast_convert_type(
        lo16 | (hi16 << jnp.uint32(16)), jnp.int32)
    logits = jnp.dot(x, gw_ref[...], preferred_element_type=jnp.float32)
    lane = lax.broadcasted_iota(jnp.int32, (T, E), 1)
    m1 = jnp.max(logits, axis=1, keepdims=True)
    e1 = jnp.min(jnp.where(logits == m1, lane, E), axis=1, keepdims=True)
    masked = jnp.where(lane == e1, -jnp.inf, logits)
    m2 = jnp.max(masked, axis=1, keepdims=True)
    e2 = jnp.min(jnp.where(masked == m2, lane, E), axis=1, keepdims=True)
    w1 = jax.nn.sigmoid(m1 - m2)
    w0_ref[...] = w1
    w1_ref[...] = 1.0 - w1

    oh1 = (lane == e1).astype(jnp.float32)
    oh2 = (lane == e2).astype(jnp.float32)
    ri = lax.broadcasted_iota(jnp.int32, (T, T), 0)
    ci = lax.broadcasted_iota(jnp.int32, (T, T), 1)
    ltri = (ci <= ri).astype(jnp.float32)
    c1 = jnp.dot(ltri, oh1, preferred_element_type=jnp.float32)
    tot1 = jnp.sum(oh1, axis=0, keepdims=True)
    c2 = jnp.dot(ltri, oh2, preferred_element_type=jnp.float32) + tot1
    tot = tot1 + jnp.sum(oh2, axis=0, keepdims=True)
    nblk = jnp.floor((tot + (BLK - 1)) / BLK)
    si = lax.broadcasted_iota(jnp.int32, (E, E), 0)
    sj = lax.broadcasted_iota(jnp.int32, (E, E), 1)
    stri = (si < sj).astype(jnp.float32)
    bstart = jnp.dot(nblk, stri, preferred_element_type=jnp.float32)
    po = bstart * BLK
    rank0 = jnp.sum(oh1 * c1, axis=1, keepdims=True) - 1.0
    rank1 = jnp.sum(oh2 * c2, axis=1, keepdims=True) - 1.0
    po1 = jnp.sum(oh1 * po, axis=1, keepdims=True)
    po2 = jnp.sum(oh2 * po, axis=1, keepdims=True)
    p0_ref[...] = (po1 + rank0).astype(jnp.int32)
    p1_ref[...] = (po2 + rank1).astype(jnp.int32)
    tot = jnp.sum(nblk, keepdims=True)
    bif = lax.broadcasted_iota(jnp.int32, (NB, 1), 0).astype(jnp.float32)
    active = (nblk > 0.0)
    ind = (bif >= bstart).astype(jnp.float32)
    be = jnp.sum(ind, axis=1, keepdims=True) - 1.0
    chg = jnp.sum(((bstart == bif) & active).astype(jnp.float32), axis=1,
                  keepdims=True)
    runcnt = jnp.sum(((bstart <= bif) & active).astype(jnp.float32), axis=1,
                     keepdims=True)
    runm1 = runcnt - 1.0
    par = runm1 - 3.0 * jnp.floor(runm1 / 3.0)
    bigv = jnp.float32(1e9)
    startsf = jnp.where(active, bstart, bigv)
    cand = jnp.where(startsf > bif, startsf, bigv)
    nmin = jnp.min(cand, axis=1, keepdims=True)
    hn = (nmin < 1e8).astype(jnp.float32)
    eiota = lax.broadcasted_iota(jnp.int32, (NB, E), 1).astype(jnp.float32)
    nxt = jnp.sum(jnp.where(cand == nmin, eiota, 0.0), axis=1, keepdims=True)
    cand2 = jnp.where(cand == nmin, bigv, cand)
    n2min = jnp.min(cand2, axis=1, keepdims=True)
    hn2 = (n2min < 1e8).astype(jnp.float32)
    nxt2 = jnp.sum(jnp.where(cand2 == n2min, eiota, 0.0), axis=1, keepdims=True)
    totb = jnp.zeros((NB, 1), jnp.float32) + tot
    tbl = jnp.concatenate([be, chg, par, nxt, hn, totb, nxt2, hn2], axis=1)
    tbl_ref[...] = tbl.astype(jnp.int32)


def _route(x, gate_w):
    return pl.pallas_call(
        _route_body,
        out_shape=(
            jax.ShapeDtypeStruct((T, 1), jnp.int32),
            jax.ShapeDtypeStruct((T, 1), jnp.int32),
            jax.ShapeDtypeStruct((T, 1), jnp.float32),
            jax.ShapeDtypeStruct((T, 1), jnp.float32),
            jax.ShapeDtypeStruct((NB, 8), jnp.int32),
            jax.ShapeDtypeStruct((T, D // 2), jnp.int32),
        ),
    )(x, gate_w)


def _ffn_body(tbl_ref, xs_ref, wg_ref, wu_ref, wd_ref, y_ref,
              xbuf, wgbuf, wubuf, wdbuf, ybuf, sx, sw, sy):
    tot = tbl_ref[0, 5]

    def wcopies(e, slot):
        h1, h2 = pl.ds(0, D // 2), pl.ds(D // 2, D // 2)
        f1, f2 = pl.ds(0, F // 2), pl.ds(F // 2, F // 2)
        return (pltpu.make_async_copy(wg_ref.at[e, h1], wgbuf.at[slot, h1], sw),
                pltpu.make_async_copy(wg_ref.at[e, h2], wgbuf.at[slot, h2], sw),
                pltpu.make_async_copy(wu_ref.at[e, h1], wubuf.at[slot, h1], sw),
                pltpu.make_async_copy(wu_ref.at[e, h2], wubuf.at[slot, h2], sw),
                pltpu.make_async_copy(wd_ref.at[e, f1], wdbuf.at[slot, f1], sw),
                pltpu.make_async_copy(wd_ref.at[e, f2], wdbuf.at[slot, f2], sw))

    def xcopy(i, slot):
        return pltpu.make_async_copy(
            xs_ref.at[pl.ds(i * BLK, BLK)], xbuf.at[slot], sx)

    def ycopy(i, slot):
        return pltpu.make_async_copy(
            ybuf.at[slot], y_ref.at[pl.ds(i * BLK, BLK)], sy)

    xcopy(0, 0).start()
    for c in wcopies(tbl_ref[0, 0], 0):
        c.start()

    @pl.when(tbl_ref[0, 4] == 1)
    def _():
        for c in wcopies(tbl_ref[0, 3], 1):
            c.start()

    def step(i, carry):
        q = lax.rem(i, 2)
        bev = tbl_ref[i, 0]
        chg = tbl_ref[i, 1]
        par = tbl_ref[i, 2]
        nxt = tbl_ref[i, 3]
        hn = tbl_ref[i, 4]

        hn2 = tbl_ref[i, 7]
        nxt2 = tbl_ref[i, 6]

        @pl.when(chg == 1)
        def _():
            for c in wcopies(bev, par):
                c.wait()

            @pl.when(hn2 == 1)
            def _():
                slot2 = jnp.where(par >= 1, par - 1, 2)
                for c in wcopies(nxt2, slot2):
                    c.start()

        xcopy(i, q).wait()

        @pl.when(i + 1 < tot)
        def _():
            xcopy(i + 1, 1 - q).start()

        @pl.when(i >= 2)
        def _():
            ycopy(i, q).wait()

        xw = xbuf[q]
        xlo = lax.bitcast_convert_type(xw << 16, jnp.float32)
        xhi = lax.bitcast_convert_type(xw & jnp.int32(-65536), jnp.float32)
        wg = wgbuf[par]
        wu = wubuf[par]
        g = (jnp.dot(xlo, wg[:D // 2], preferred_element_type=jnp.float32) +
             jnp.dot(xhi, wg[D // 2:], preferred_element_type=jnp.float32))
        u = (jnp.dot(xlo, wu[:D // 2], preferred_element_type=jnp.float32) +
             jnp.dot(xhi, wu[D // 2:], preferred_element_type=jnp.float32))
        h = g * jax.nn.sigmoid(g) * u
        ybuf[q, ...] = jnp.dot(h, wdbuf[par], preferred_element_type=jnp.float32)
        ycopy(i, q).start()
        return carry

    lax.fori_loop(0, tot, step, 0)
    ycopy(0, 0).wait()
    ycopy(0, 1).wait()


def _ffn(tbl, xs, w_gate, w_up, w_down):
    grid_spec = pltpu.PrefetchScalarGridSpec(
        num_scalar_prefetch=1,
        grid=(1,),
        in_specs=[
            pl.BlockSpec(memory_space=pl.ANY),
            pl.BlockSpec(memory_space=pl.ANY),
            pl.BlockSpec(memory_space=pl.ANY),
            pl.BlockSpec(memory_space=pl.ANY),
        ],
        out_specs=pl.BlockSpec(memory_space=pl.ANY),
        scratch_shapes=[
            pltpu.VMEM((2, BLK, D // 2), jnp.int32),
            pltpu.VMEM((3, D, F), jnp.float32),
            pltpu.VMEM((3, D, F), jnp.float32),
            pltpu.VMEM((3, F, D), jnp.float32),
            pltpu.VMEM((2, BLK, D), jnp.float32),
            pltpu.SemaphoreType.DMA,
            pltpu.SemaphoreType.DMA,
            pltpu.SemaphoreType.DMA,
        ],
    )
    return pl.pallas_call(
        _ffn_body,
        grid_spec=grid_spec,
        out_shape=jax.ShapeDtypeStruct((R, D), jnp.float32),
    )(tbl, xs, w_gate, w_up, w_down)


@functools.cache
def _get_scatter_x():
    mesh = plsc.VectorSubcoreMesh(core_axis_name="c", subcore_axis_name="s")
    return pl.kernel(
        _scatter_x_body,
        mesh=mesh,
        out_type=jax.ShapeDtypeStruct((R, D // 2), jnp.int32),
        scratch_types=[
            pltpu.VMEM((SCH,), jnp.int32),
            pltpu.VMEM((SCH, D // 2), jnp.int32),
            pltpu.SemaphoreType.DMA,
        ],
    )


def _scatter_x_body(x_hbm, p0_hbm, p1_hbm, xs_hbm, idx_v, rows_v, sem):
    k = lax.axis_index("c")
    t0 = lax.axis_index("s") * SCH

    @pl.when(k == 0)
    def _():
        pltpu.sync_copy(p0_hbm.at[pl.ds(t0, SCH)], idx_v)

    @pl.when(k == 1)
    def _():
        pltpu.sync_copy(p1_hbm.at[pl.ds(t0, SCH)], idx_v)

    pltpu.sync_copy(x_hbm.at[pl.ds(t0, SCH)], rows_v)
    pltpu.async_copy(rows_v, xs_hbm.at[idx_v], sem).wait()


@functools.cache
def _get_combine():
    mesh = plsc.VectorSubcoreMesh(core_axis_name="c", subcore_axis_name="s")
    return pl.kernel(
        _combine_body,
        mesh=mesh,
        out_type=jax.ShapeDtypeStruct((T, D), jnp.float32),
        scratch_types=[
            pltpu.VMEM((CCH,), jnp.int32),
            pltpu.VMEM((CCH,), jnp.int32),
            pltpu.VMEM((CCH,), jnp.int32),
            pltpu.VMEM((CCH,), jnp.int32),
            pltpu.VMEM((TPW,), jnp.float32),
            pltpu.VMEM((TPW,), jnp.float32),
            pltpu.VMEM((CCH, D), jnp.float32),
            pltpu.VMEM((CCH, D), jnp.float32),
            pltpu.VMEM((CCH, D), jnp.float32),
            pltpu.VMEM((CCH, D), jnp.float32),
            pltpu.SemaphoreType.DMA,
            pltpu.SemaphoreType.DMA,
        ],
    )


def _combine_body(y_hbm, p0_hbm, p1_hbm, w0_hbm, w1_hbm, out_hbm,
                  i0a_v, i1a_v, i0b_v, i1b_v, w0_v, w1_v,
                  r0a_v, r1a_v, r0b_v, r1b_v, sem_a, sem_b):
    wid = lax.axis_index("c") * _NS + lax.axis_index("s")
    tb = wid * TPW
    pltpu.sync_copy(w0_hbm.at[pl.ds(tb, TPW)], w0_v)
    pltpu.sync_copy(w1_hbm.at[pl.ds(tb, TPW)], w1_v)
    pltpu.sync_copy(p0_hbm.at[pl.ds(tb, CCH)], i0a_v)
    pltpu.sync_copy(p1_hbm.at[pl.ds(tb, CCH)], i1a_v)
    pltpu.sync_copy(p0_hbm.at[pl.ds(tb + CCH, CCH)], i0b_v)
    pltpu.sync_copy(p1_hbm.at[pl.ds(tb + CCH, CCH)], i1b_v)
    a0 = pltpu.async_copy(y_hbm.at[i0a_v], r0a_v, sem_a)
    a1 = pltpu.async_copy(y_hbm.at[i1a_v], r1a_v, sem_a)
    b0 = pltpu.async_copy(y_hbm.at[i0b_v], r0b_v, sem_b)
    b1 = pltpu.async_copy(y_hbm.at[i1b_v], r1b_v, sem_b)

    def weighted_sum(r0_v, r1_v, gbase):
        for g in range(CCH // _L):
            wv0 = w0_v[pl.ds((gbase + g) * _L, _L)]
            wv1 = w1_v[pl.ds((gbase + g) * _L, _L)]

            def body(lane, carry, wv0=wv0, wv1=wv1, g=g):
                iv = jnp.full((_L,), lane, jnp.int32)
                w0s = wv0.at[iv].get(mode="promise_in_bounds")
                w1s = wv1.at[iv].get(mode="promise_in_bounds")
                i = g * _L + lane
                for j in range(D // _L):
                    sl = pl.ds(j * _L, _L)
                    r0_v[i, sl] = w0s * r0_v[i, sl] + w1s * r1_v[i, sl]
                return carry

            lax.fori_loop(0, _L, body, 0, unroll=2)

    a0.wait()
    a1.wait()
    weighted_sum(r0a_v, r1a_v, 0)
    st_a = pltpu.async_copy(r0a_v, out_hbm.at[pl.ds(tb, CCH)], sem_a)
    b0.wait()
    b1.wait()
    weighted_sum(r0b_v, r1b_v, CCH // _L)
    st_a.wait()
    pltpu.sync_copy(r0b_v, out_hbm.at[pl.ds(tb + CCH, CCH)])


def kernel(hidden_states, gate_w, w_gate, w_up, w_down):
    b, s, d = hidden_states.shape
    x = hidden_states.reshape(-1, d)
    p0, p1, w0, w1, tbl, xb = _route(x, gate_w)
    p0, p1 = p0.reshape(T), p1.reshape(T)
    w0, w1 = w0.reshape(T), w1.reshape(T)
    xs = _get_scatter_x()(xb, p0, p1)
    ys = _ffn(tbl, xs, w_gate, w_up, w_down)
    out = _get_combine()(ys, p0, p1, w0, w1)
    return out.reshape(b, s, d)


# speedup vs baseline: 2.1696x; 1.0432x over previous
import functools

import jax
import jax.numpy as jnp
from jax import lax
from jax.experimental import pallas as pl
from jax.experimental.pallas import tpu as pltpu
from jax.experimental.pallas import tpu_sc as plsc

T = 2048
D = 768
F = 512
E = 16
TOPK = 2
BLK = 384
NB = 25
R = NB * BLK
P = T * TOPK

_NC, _NS, _L = 2, 16, 16
NW = _NC * _NS
SCH = T // _NS
TPW = T // NW
CCH = TPW // 2


def _route_body(x_ref, gw_ref, p0_ref, p1_ref, w0_ref, w1_ref, tbl_ref, xb_ref):
    x = x_ref[...]
    xu = lax.bitcast_convert_type(x, jnp.uint32)
    rnd = (xu + jnp.uint32(0x7FFF) +
           ((xu >> jnp.uint32(16)) & jnp.uint32(1))) >> jnp.uint32(16)
    lo16 = rnd[:, :D // 2]
    hi16 = rnd[:, D // 2:]
    xb_ref[...] = lax.bitcast_convert_type(
        lo16 | (hi16 << jnp.uint32(16)), jnp.int32)
    logits = jnp.dot(x, gw_ref[...], preferred_element_type=jnp.float32)
    lane = lax.broadcasted_iota(jnp.int32, (T, E), 1)
    m1 = jnp.max(logits, axis=1, keepdims=True)
    e1 = jnp.min(jnp.where(logits == m1, lane, E), axis=1, keepdims=True)
    masked = jnp.where(lane == e1, -jnp.inf, logits)
    m2 = jnp.max(masked, axis=1, keepdims=True)
    e2 = jnp.min(jnp.where(masked == m2, lane, E), axis=1, keepdims=True)
    w1 = jax.nn.sigmoid(m1 - m2)
    w0_ref[...] = w1
    w1_ref[...] = 1.0 - w1

    oh1 = (lane == e1).astype(jnp.float32)
    oh2 = (lane == e2).astype(jnp.float32)
    ri = lax.broadcasted_iota(jnp.int32, (T, T), 0)
    ci = lax.broadcasted_iota(jnp.int32, (T, T), 1)
    ltri = (ci <= ri).astype(jnp.float32)
    c1 = jnp.dot(ltri, oh1, preferred_element_type=jnp.float32)
    tot1 = jnp.sum(oh1, axis=0, keepdims=True)
    c2 = jnp.dot(ltri, oh2, preferred_element_type=jnp.float32) + tot1
    tot = tot1 + jnp.sum(oh2, axis=0, keepdims=True)
    nblk = jnp.floor((tot + (BLK - 1)) / BLK)
    si = lax.broadcasted_iota(jnp.int32, (E, E), 0)
    sj = lax.broadcasted_iota(jnp.int32, (E, E), 1)
    stri = (si < sj).astype(jnp.float32)
    bstart = jnp.dot(nblk, stri, preferred_element_type=jnp.float32)
    po = bstart * BLK
    rank0 = jnp.sum(oh1 * c1, axis=1, keepdims=True) - 1.0
    rank1 = jnp.sum(oh2 * c2, axis=1, keepdims=True) - 1.0
    po1 = jnp.sum(oh1 * po, axis=1, keepdims=True)
    po2 = jnp.sum(oh2 * po, axis=1, keepdims=True)
    p0_ref[...] = (po1 + rank0).astype(jnp.int32)
    p1_ref[...] = (po2 + rank1).astype(jnp.int32)
    tot = jnp.sum(nblk, keepdims=True)
    bif = lax.broadcasted_iota(jnp.int32, (NB, 1), 0).astype(jnp.float32)
    active = (nblk > 0.0)
    ind = (bif >= bstart).astype(jnp.float32)
    be = jnp.sum(ind, axis=1, keepdims=True) - 1.0
    chg = jnp.sum(((bstart == bif) & active).astype(jnp.float32), axis=1,
                  keepdims=True)
    runcnt = jnp.sum(((bstart <= bif) & active).astype(jnp.float32), axis=1,
                     keepdims=True)
    runm1 = runcnt - 1.0
    par = runm1 - 3.0 * jnp.floor(runm1 / 3.0)
    bigv = jnp.float32(1e9)
    startsf = jnp.where(active, bstart, bigv)
    cand = jnp.where(startsf > bif, startsf, bigv)
    nmin = jnp.min(cand, axis=1, keepdims=True)
    hn = (nmin < 1e8).astype(jnp.float32)
    eiota = lax.broadcasted_iota(jnp.int32, (NB, E), 1).astype(jnp.float32)
    nxt = jnp.sum(jnp.where(cand == nmin, eiota, 0.0), axis=1, keepdims=True)
    cand2 = jnp.where(cand == nmin, bigv, cand)
    n2min = jnp.min(cand2, axis=1, keepdims=True)
    hn2 = (n2min < 1e8).astype(jnp.float32)
    nxt2 = jnp.sum(jnp.where(cand2 == n2min, eiota, 0.0), axis=1, keepdims=True)
    totb = jnp.zeros((NB, 1), jnp.float32) + tot
    tbl = jnp.concatenate([be, chg, par, nxt, hn, totb, nxt2, hn2], axis=1)
    tbl_ref[...] = tbl.astype(jnp.int32)


def _route(x, gate_w):
    return pl.pallas_call(
        _route_body,
        out_shape=(
            jax.ShapeDtypeStruct((T, 1), jnp.int32),
            jax.ShapeDtypeStruct((T, 1), jnp.int32),
            jax.ShapeDtypeStruct((T, 1), jnp.float32),
            jax.ShapeDtypeStruct((T, 1), jnp.float32),
            jax.ShapeDtypeStruct((NB, 8), jnp.int32),
            jax.ShapeDtypeStruct((T, D // 2), jnp.int32),
        ),
    )(x, gate_w)


def _ffn_body(tbl_ref, xs_ref, wg_ref, wu_ref, wd_ref, y_ref,
              xbuf, wgbuf, wubuf, wdbuf, ybuf, sx, sw, sy):
    tot = tbl_ref[0, 5]

    def wcopies(e, slot):
        h1, h2 = pl.ds(0, D // 2), pl.ds(D // 2, D // 2)
        f1, f2 = pl.ds(0, F // 2), pl.ds(F // 2, F // 2)
        return (pltpu.make_async_copy(wg_ref.at[e, h1], wgbuf.at[slot, h1], sw),
                pltpu.make_async_copy(wg_ref.at[e, h2], wgbuf.at[slot, h2], sw),
                pltpu.make_async_copy(wu_ref.at[e, h1], wubuf.at[slot, h1], sw),
                pltpu.make_async_copy(wu_ref.at[e, h2], wubuf.at[slot, h2], sw),
                pltpu.make_async_copy(wd_ref.at[e, f1], wdbuf.at[slot, f1], sw),
                pltpu.make_async_copy(wd_ref.at[e, f2], wdbuf.at[slot, f2], sw))

    def xcopy(i, slot):
        return pltpu.make_async_copy(
            xs_ref.at[pl.ds(i * BLK, BLK)], xbuf.at[slot], sx)

    def ycopy(i, slot):
        return pltpu.make_async_copy(
            ybuf.at[slot], y_ref.at[pl.ds(i * BLK, BLK)], sy)

    xcopy(0, 0).start()
    for c in wcopies(tbl_ref[0, 0], 0):
        c.start()

    @pl.when(tbl_ref[0, 4] == 1)
    def _():
        for c in wcopies(tbl_ref[0, 3], 1):
            c.start()

    def step(i, carry):
        q = lax.rem(i, 2)
        bev = tbl_ref[i, 0]
        chg = tbl_ref[i, 1]
        par = tbl_ref[i, 2]
        nxt = tbl_ref[i, 3]
        hn = tbl_ref[i, 4]

        hn2 = tbl_ref[i, 7]
        nxt2 = tbl_ref[i, 6]

        @pl.when(chg == 1)
        def _():
            for c in wcopies(bev, par):
                c.wait()

            @pl.when(hn2 == 1)
            def _():
                slot2 = jnp.where(par >= 1, par - 1, 2)
                for c in wcopies(nxt2, slot2):
                    c.start()

        xcopy(i, q).wait()

        @pl.when(i + 1 < tot)
        def _():
            xcopy(i + 1, 1 - q).start()

        @pl.when(i >= 2)
        def _():
            ycopy(i, q).wait()

        xw = xbuf[q]
        xlo = lax.bitcast_convert_type(xw << 16, jnp.float32)
        xhi = lax.bitcast_convert_type(xw & jnp.int32(-65536), jnp.float32)
        wg = wgbuf[par]
        wu = wubuf[par]
        g = (jnp.dot(xlo, wg[:D // 2], preferred_element_type=jnp.float32) +
             jnp.dot(xhi, wg[D // 2:], preferred_element_type=jnp.float32))
        u = (jnp.dot(xlo, wu[:D // 2], preferred_element_type=jnp.float32) +
             jnp.dot(xhi, wu[D // 2:], preferred_element_type=jnp.float32))
        h = g * jax.nn.sigmoid(g) * u
        ybuf[q, ...] = jnp.dot(h, wdbuf[par], preferred_element_type=jnp.float32)
        ycopy(i, q).start()
        return carry

    lax.fori_loop(0, tot, step, 0)
    ycopy(0, 0).wait()
    ycopy(0, 1).wait()


def _ffn(tbl, xs, w_gate, w_up, w_down):
    grid_spec = pltpu.PrefetchScalarGridSpec(
        num_scalar_prefetch=1,
        grid=(1,),
        in_specs=[
            pl.BlockSpec(memory_space=pl.ANY),
            pl.BlockSpec(memory_space=pl.ANY),
            pl.BlockSpec(memory_space=pl.ANY),
            pl.BlockSpec(memory_space=pl.ANY),
        ],
        out_specs=pl.BlockSpec(memory_space=pl.ANY),
        scratch_shapes=[
            pltpu.VMEM((2, BLK, D // 2), jnp.int32),
            pltpu.VMEM((3, D, F), jnp.float32),
            pltpu.VMEM((3, D, F), jnp.float32),
            pltpu.VMEM((3, F, D), jnp.float32),
            pltpu.VMEM((2, BLK, D), jnp.float32),
            pltpu.SemaphoreType.DMA,
            pltpu.SemaphoreType.DMA,
            pltpu.SemaphoreType.DMA,
        ],
    )
    return pl.pallas_call(
        _ffn_body,
        grid_spec=grid_spec,
        out_shape=jax.ShapeDtypeStruct((R, D), jnp.float32),
    )(tbl, xs, w_gate, w_up, w_down)


@functools.cache
def _get_scatter_x():
    mesh = plsc.VectorSubcoreMesh(core_axis_name="c", subcore_axis_name="s")
    return pl.kernel(
        _scatter_x_body,
        mesh=mesh,
        out_type=jax.ShapeDtypeStruct((R, D // 2), jnp.int32),
        scratch_types=[
            pltpu.VMEM((SCH,), jnp.int32),
            pltpu.VMEM((SCH, D // 2), jnp.int32),
            pltpu.SemaphoreType.DMA,
        ],
    )


def _scatter_x_body(x_hbm, p0_hbm, p1_hbm, xs_hbm, idx_v, rows_v, sem):
    k = lax.axis_index("c")
    t0 = lax.axis_index("s") * SCH

    @pl.when(k == 0)
    def _():
        pltpu.sync_copy(p0_hbm.at[pl.ds(t0, SCH)], idx_v)

    @pl.when(k == 1)
    def _():
        pltpu.sync_copy(p1_hbm.at[pl.ds(t0, SCH)], idx_v)

    pltpu.sync_copy(x_hbm.at[pl.ds(t0, SCH)], rows_v)
    pltpu.async_copy(rows_v, xs_hbm.at[idx_v], sem).wait()


@functools.cache
def _get_combine():
    mesh = plsc.VectorSubcoreMesh(core_axis_name="c", subcore_axis_name="s")
    return pl.kernel(
        _combine_body,
        mesh=mesh,
        out_type=jax.ShapeDtypeStruct((T, D), jnp.float32),
        scratch_types=[
            pltpu.VMEM((CCH,), jnp.int32),
            pltpu.VMEM((CCH,), jnp.int32),
            pltpu.VMEM((CCH,), jnp.int32),
            pltpu.VMEM((CCH,), jnp.int32),
            pltpu.VMEM((TPW,), jnp.float32),
            pltpu.VMEM((TPW,), jnp.float32),
            pltpu.VMEM((CCH, D), jnp.float32),
            pltpu.VMEM((CCH, D), jnp.float32),
            pltpu.VMEM((CCH, D), jnp.float32),
            pltpu.VMEM((CCH, D), jnp.float32),
            pltpu.SemaphoreType.DMA,
            pltpu.SemaphoreType.DMA,
        ],
    )


def _combine_body(y_hbm, p0_hbm, p1_hbm, w0_hbm, w1_hbm, out_hbm,
                  i0a_v, i1a_v, i0b_v, i1b_v, w0_v, w1_v,
                  r0a_v, r1a_v, r0b_v, r1b_v, sem_a, sem_b):
    wid = lax.axis_index("c") * _NS + lax.axis_index("s")
    tb = wid * TPW
    pltpu.sync_copy(w0_hbm.at[pl.ds(tb, TPW)], w0_v)
    pltpu.sync_copy(w1_hbm.at[pl.ds(tb, TPW)], w1_v)
    pltpu.sync_copy(p0_hbm.at[pl.ds(tb, CCH)], i0a_v)
    pltpu.sync_copy(p1_hbm.at[pl.ds(tb, CCH)], i1a_v)
    pltpu.sync_copy(p0_hbm.at[pl.ds(tb + CCH, CCH)], i0b_v)
    pltpu.sync_copy(p1_hbm.at[pl.ds(tb + CCH, CCH)], i1b_v)
    a0 = pltpu.async_copy(y_hbm.at[i0a_v], r0a_v, sem_a)
    a1 = pltpu.async_copy(y_hbm.at[i1a_v], r1a_v, sem_a)
    b0 = pltpu.async_copy(y_hbm.at[i0b_v], r0b_v, sem_b)
    b1 = pltpu.async_copy(y_hbm.at[i1b_v], r1b_v, sem_b)

    def weighted_sum(r0_v, r1_v, gbase):
        for g in range(CCH // _L):
            wv0 = w0_v[pl.ds((gbase + g) * _L, _L)]
            wv1 = w1_v[pl.ds((gbase + g) * _L, _L)]

            def body(lane, carry, wv0=wv0, wv1=wv1, g=g):
                iv = jnp.full((_L,), lane, jnp.int32)
                w0s = wv0.at[iv].get(mode="promise_in_bounds")
                w1s = wv1.at[iv].get(mode="promise_in_bounds")
                i = g * _L + lane
                for j in range(D // _L):
                    sl = pl.ds(j * _L, _L)
                    r0_v[i, sl] = w0s * r0_v[i, sl] + w1s * r1_v[i, sl]
                return carry

            lax.fori_loop(0, _L, body, 0, unroll=2)

    a0.wait()
    a1.wait()
    weighted_sum(r0a_v, r1a_v, 0)
    st_a = pltpu.async_copy(r0a_v, out_hbm.at[pl.ds(tb, CCH)], sem_a)
    b0.wait()
    b1.wait()
    weighted_sum(r0b_v, r1b_v, CCH // _L)
    st_a.wait()
    pltpu.sync_copy(r0b_v, out_hbm.at[pl.ds(tb + CCH, CCH)])


def kernel(hidden_states, gate_w, w_gate, w_up, w_down):
    b, s, d = hidden_states.shape
    x = hidden_states.reshape(-1, d)
    p0, p1, w0, w1, tbl, xb = _route(x, gate_w)
    p0, p1 = p0.reshape(T), p1.reshape(T)
    w0, w1 = w0.reshape(T), w1.reshape(T)
    xs = _get_scatter_x()(xb, p0, p1)
    ys = _ffn(tbl, xs, w_gate, w_up, w_down)
    out = _get_combine()(ys, p0, p1, w0, w1)
    return out.reshape(b, s, d)
